# Initial kernel scaffold; baseline (speedup 1.0000x reference)
import jax
import jax.numpy as jnp
from jax import lax
from jax.experimental import pallas as pl
from jax.experimental.pallas import tpu as pltpu
from jax.experimental.pallas import tpu_sc as plsc

N = 10000
NP = 10240
E = 160000
EP = 163840
D = 300
DP = 384
DH = 192
H2 = 600
HP = 768
G = 128
L = 5
T = 128
BN_ROWS = 1024
NBLK = NP // BN_ROWS
CH = 128
EDGES_PER_SUB = EP // 16
ECHUNKS = EDGES_PER_SUB // CH
ROWS_PER_SUB = NP // 16
RCHUNKS = ROWS_PER_SUB // CH

_MESH = dict(core_axis_name="c", subcore_axis_name="s")


def _zero_rows(buf):
    z = jnp.zeros((16,), jnp.float32)

    def row(r, _):
        for j in range(DH // 16):
            buf[r, pl.ds(j * 16, 16)] = z
        return 0

    lax.fori_loop(0, CH, row, 0)


def _sc_embed(atlo, athi, xidx):
    def body(atlo_r, athi_r, xidx_r, olo_r, ohi_r, idxv, gbuf, accbuf, sem):
        c = lax.axis_index("c")
        s = lax.axis_index("s")

        def process(tab_ref, out_ref):
            def chunk(k, _):
                rowbase = (s * RCHUNKS + k) * CH
                for f in range(9):
                    pltpu.sync_copy(xidx_r.at[pl.ds(f * NP + rowbase, CH)], idxv)
                    pltpu.async_copy(tab_ref.at[idxv], gbuf, sem).wait()

                    if f == 0:
                        def rowcp(r, _):
                            for j in range(DH // 16):
                                sl = pl.ds(j * 16, 16)
                                accbuf[r, sl] = gbuf[r, sl]
                            return 0
                        lax.fori_loop(0, CH, rowcp, 0)
                    else:
                        def rowadd(r, _):
                            for j in range(DH // 16):
                                sl = pl.ds(j * 16, 16)
                                accbuf[r, sl] = accbuf[r, sl] + gbuf[r, sl]
                            return 0
                        lax.fori_loop(0, CH, rowadd, 0)
                pltpu.sync_copy(accbuf, out_ref.at[pl.ds(rowbase, CH)])
                return 0

            lax.fori_loop(0, RCHUNKS, chunk, 0)

        @pl.when(c == 0)
        def _():
            process(atlo_r, olo_r)

        @pl.when(c == 1)
        def _():
            process(athi_r, ohi_r)

    f = pl.kernel(
        body,
        out_type=[
            jax.ShapeDtypeStruct((NP, DH), jnp.float32),
            jax.ShapeDtypeStruct((NP, DH), jnp.float32),
        ],
        mesh=plsc.VectorSubcoreMesh(**_MESH),
        scratch_types=[
            pltpu.VMEM((CH,), jnp.int32),
            pltpu.VMEM((CH, DH), jnp.float32),
            pltpu.VMEM((CH, DH), jnp.float32),
            pltpu.SemaphoreType.DMA,
        ],
    )
    return f(atlo, athi, xidx)


def _sc_edge(hlo, hhi, ctlo, cthi, srcp, dstp, codep):
    def body(hlo_r, hhi_r, ctlo_r, cthi_r, src_r, dst_r, code_r,
             alo_r, ahi_r, srcv, dstv, codev, hbuf, ebuf, tmpbuf,
             aggr_sh, sem1, sem2):
        c = lax.axis_index("c")
        s = lax.axis_index("s")

        _zero_rows(tmpbuf)
        for k in range(RCHUNKS):
            pltpu.sync_copy(tmpbuf, aggr_sh.at[pl.ds((s * RCHUNKS + k) * CH, CH)])
        plsc.subcore_barrier()

        def process(h_ref, ct_ref, out_ref):
            def chunk(k, _):
                base = s * EDGES_PER_SUB + k * CH
                pltpu.sync_copy(src_r.at[pl.ds(base, CH)], srcv)
                pltpu.sync_copy(code_r.at[pl.ds(base, CH)], codev)
                pltpu.sync_copy(dst_r.at[pl.ds(base, CH)], dstv)
                cp1 = pltpu.async_copy(h_ref.at[srcv], hbuf, sem1)
                cp2 = pltpu.async_copy(ct_ref.at[codev], ebuf, sem2)
                cp1.wait()
                cp2.wait()

                def row(r, _):
                    for j in range(DH // 16):
                        sl = pl.ds(j * 16, 16)
                        ebuf[r, sl] = jnp.maximum(hbuf[r, sl] + ebuf[r, sl], 0.0)
                    return 0

                lax.fori_loop(0, CH, row, 0)
                pltpu.sync_copy(ebuf, aggr_sh.at[dstv], add=True)
                return 0

            lax.fori_loop(0, ECHUNKS, chunk, 0)
            plsc.subcore_barrier()
            for k in range(RCHUNKS):
                r0 = (s * RCHUNKS + k) * CH
                pltpu.sync_copy(aggr_sh.at[pl.ds(r0, CH)], tmpbuf)
                pltpu.sync_copy(tmpbuf, out_ref.at[pl.ds(r0, CH)])

        @pl.when(c == 0)
        def _():
            process(hlo_r, ctlo_r, alo_r)

        @pl.when(c == 1)
        def _():
            process(hhi_r, cthi_r, ahi_r)

    f = pl.kernel(
        body,
        out_type=[
            jax.ShapeDtypeStruct((NP, DH), jnp.float32),
            jax.ShapeDtypeStruct((NP, DH), jnp.float32),
        ],
        mesh=plsc.VectorSubcoreMesh(**_MESH),
        scratch_types=[
            pltpu.VMEM((CH,), jnp.int32),
            pltpu.VMEM((CH,), jnp.int32),
            pltpu.VMEM((CH,), jnp.int32),
            pltpu.VMEM((CH, DH), jnp.float32),
            pltpu.VMEM((CH, DH), jnp.float32),
            pltpu.VMEM((CH, DH), jnp.float32),
            pltpu.VMEM_SHARED((NP, DH), jnp.float32),
            pltpu.SemaphoreType.DMA,
            pltpu.SemaphoreType.DMA,
        ],
    )
    return f(hlo, hhi, ctlo, cthi, srcp, dstp, codep)


def _onehot(b):
    return (b[:, None] == lax.broadcasted_iota(jnp.int32, (BN_ROWS, G), 1)
            ).astype(jnp.float32)


def _stats8(x, mask):
    xm = jnp.where(mask, x, 0.0)
    s0 = jnp.sum(xm, axis=0, keepdims=True)
    s1 = jnp.sum(xm * xm, axis=0, keepdims=True)
    return jnp.concatenate(
        [s0, s1, jnp.zeros((6, x.shape[1]), jnp.float32)], axis=0)


def _scale_shift(stats, g, b):
    m = stats[0, :] * (1.0 / N)
    v = stats[1, :] * (1.0 / N) - m * m
    scale = g * lax.rsqrt(v + 1e-5)
    return scale, b - m * scale


def _pass_a(hlo, hhi, alo, ahi, w1, vecs, batch3):
    def body(hlo_r, hhi_r, alo_r, ahi_r, w1_r, vecs_r, b3_r, z1_r, st_r, s_r):
        i = pl.program_id(0)
        hin = jnp.concatenate([hlo_r[...], hhi_r[...]], axis=1)
        aggr = jnp.concatenate([alo_r[...], ahi_r[...]], axis=1)
        z = hin * vecs_r[1, 0] + aggr
        z1 = jnp.dot(z, w1_r[...], preferred_element_type=jnp.float32)
        z1 = z1 + vecs_r[0, :]
        z1_r[...] = z1
        oh = _onehot(b3_r[0, 0, :])
        sacc = lax.dot_general(oh, hin, (((0,), (0,)), ((), ())),
                               preferred_element_type=jnp.float32)
        rowid = i * BN_ROWS + lax.broadcasted_iota(jnp.int32, (BN_ROWS, 1), 0)
        st = _stats8(z1, rowid < N)

        @pl.when(i == 0)
        def _():
            st_r[...] = st
            s_r[...] = sacc

        @pl.when(i > 0)
        def _():
            st_r[...] = st_r[...] + st
            s_r[...] = s_r[...] + sacc

    return pl.pallas_call(
        body,
        grid=(NBLK,),
        in_specs=[
            pl.BlockSpec((BN_ROWS, DH), lambda i: (i, 0)),
            pl.BlockSpec((BN_ROWS, DH), lambda i: (i, 0)),
            pl.BlockSpec((BN_ROWS, DH), lambda i: (i, 0)),
            pl.BlockSpec((BN_ROWS, DH), lambda i: (i, 0)),
            pl.BlockSpec((DP, HP), lambda i: (0, 0)),
            pl.BlockSpec((8, HP), lambda i: (0, 0)),
            pl.BlockSpec((1, 1, BN_ROWS), lambda i: (i, 0, 0)),
        ],
        out_specs=[
            pl.BlockSpec((BN_ROWS, HP), lambda i: (i, 0)),
            pl.BlockSpec((8, HP), lambda i: (0, 0)),
            pl.BlockSpec((G, DP), lambda i: (0, 0)),
        ],
        out_shape=[
            jax.ShapeDtypeStruct((NP, HP), jnp.float32),
            jax.ShapeDtypeStruct((8, HP), jnp.float32),
            jax.ShapeDtypeStruct((G, DP), jnp.float32),
        ],
    )(hlo, hhi, alo, ahi, w1, vecs, batch3)


def _pass_b(z1, st1, gb, w2, vecs):
    def body(z1_r, st_r, gb_r, w2_r, vecs_r, z2_r, st2_r):
        i = pl.program_id(0)
        scale, shift = _scale_shift(st_r[...], gb_r[0, :], gb_r[1, :])
        a = jnp.maximum(z1_r[...] * scale + shift, 0.0)
        z2 = jnp.dot(a, w2_r[...], preferred_element_type=jnp.float32)
        z2 = z2 + vecs_r[0, :]
        z2_r[...] = z2
        rowid = i * BN_ROWS + lax.broadcasted_iota(jnp.int32, (BN_ROWS, 1), 0)
        st = _stats8(z2, rowid < N)

        @pl.when(i == 0)
        def _():
            st2_r[...] = st

        @pl.when(i > 0)
        def _():
            st2_r[...] = st2_r[...] + st

    return pl.pallas_call(
        body,
        grid=(NBLK,),
        in_specs=[
            pl.BlockSpec((BN_ROWS, HP), lambda i: (i, 0)),
            pl.BlockSpec((8, HP), lambda i: (0, 0)),
            pl.BlockSpec((8, HP), lambda i: (0, 0)),
            pl.BlockSpec((HP, DP), lambda i: (0, 0)),
            pl.BlockSpec((8, DP), lambda i: (0, 0)),
        ],
        out_specs=[
            pl.BlockSpec((BN_ROWS, DP), lambda i: (i, 0)),
            pl.BlockSpec((8, DP), lambda i: (0, 0)),
        ],
        out_shape=[
            jax.ShapeDtypeStruct((NP, DP), jnp.float32),
            jax.ShapeDtypeStruct((8, DP), jnp.float32),
        ],
    )(z1, st1, gb, w2, vecs)


def _pass_c(z2, st2, gb, vn, batch3):
    def body(z2_r, st_r, gb_r, vn_r, b3_r, olo_r, ohi_r):
        scale, shift = _scale_shift(st_r[...], gb_r[0, :], gb_r[1, :])
        h = jnp.maximum(z2_r[...] * scale + shift, 0.0)
        oh = _onehot(b3_r[0, 0, :])
        hin = h + jnp.dot(oh, vn_r[...], preferred_element_type=jnp.float32)
        olo_r[...] = hin[:, :DH]
        ohi_r[...] = hin[:, DH:]

    return pl.pallas_call(
        body,
        grid=(NBLK,),
        in_specs=[
            pl.BlockSpec((BN_ROWS, DP), lambda i: (i, 0)),
            pl.BlockSpec((8, DP), lambda i: (0, 0)),
            pl.BlockSpec((8, DP), lambda i: (0, 0)),
            pl.BlockSpec((G, DP), lambda i: (0, 0)),
            pl.BlockSpec((1, 1, BN_ROWS), lambda i: (i, 0, 0)),
        ],
        out_specs=[
            pl.BlockSpec((BN_ROWS, DH), lambda i: (i, 0)),
            pl.BlockSpec((BN_ROWS, DH), lambda i: (i, 0)),
        ],
        out_shape=[
            jax.ShapeDtypeStruct((NP, DH), jnp.float32),
            jax.ShapeDtypeStruct((NP, DH), jnp.float32),
        ],
    )(z2, st2, gb, vn, batch3)


def _pass_c_last(z2, st2, gb, batch3):
    def body(z2_r, st_r, gb_r, b3_r, s5_r):
        i = pl.program_id(0)
        scale, shift = _scale_shift(st_r[...], gb_r[0, :], gb_r[1, :])
        h = z2_r[...] * scale + shift
        oh = _onehot(b3_r[0, 0, :])
        sacc = lax.dot_general(oh, h, (((0,), (0,)), ((), ())),
                               preferred_element_type=jnp.float32)

        @pl.when(i == 0)
        def _():
            s5_r[...] = sacc

        @pl.when(i > 0)
        def _():
            s5_r[...] = s5_r[...] + sacc

    return pl.pallas_call(
        body,
        grid=(NBLK,),
        in_specs=[
            pl.BlockSpec((BN_ROWS, DP), lambda i: (i, 0)),
            pl.BlockSpec((8, DP), lambda i: (0, 0)),
            pl.BlockSpec((8, DP), lambda i: (0, 0)),
            pl.BlockSpec((1, 1, BN_ROWS), lambda i: (i, 0, 0)),
        ],
        out_specs=[pl.BlockSpec((G, DP), lambda i: (0, 0))],
        out_shape=[jax.ShapeDtypeStruct((G, DP), jnp.float32)],
    )(z2, st2, gb, batch3)


def _vn_mlp(s_in, vn_in, w1, vecs1, w2, vecs2):
    def body(s_r, vn_r, w1_r, v1_r, w2_r, v2_r, out_r):
        svn = s_r[...] + vn_r[...]
        u = jnp.dot(svn, w1_r[...], preferred_element_type=jnp.float32)
        u = u + v1_r[0, :]
        m = jnp.mean(u, axis=0)
        v = jnp.mean(u * u, axis=0) - m * m
        u = jnp.maximum(v1_r[1, :] * (u - m) * lax.rsqrt(v + 1e-5)
                        + v1_r[2, :], 0.0)
        u2 = jnp.dot(u, w2_r[...], preferred_element_type=jnp.float32)
        u2 = u2 + v2_r[0, :]
        m2 = jnp.mean(u2, axis=0)
        vv2 = jnp.mean(u2 * u2, axis=0) - m2 * m2
        out_r[...] = jnp.maximum(
            v2_r[1, :] * (u2 - m2) * lax.rsqrt(vv2 + 1e-5) + v2_r[2, :], 0.0)

    return pl.pallas_call(
        body,
        out_shape=jax.ShapeDtypeStruct((G, DP), jnp.float32),
    )(s_in, vn_in, w1, vecs1, w2, vecs2)


def _final(ss, batch3, wp, vecs):
    def body(ss_r, b3_r, wp_r, vecs_r, emb_r, log_r):
        pooled = jnp.sum(ss_r[...], axis=0)
        cnt = jnp.zeros((G,), jnp.float32)
        for i in range(NBLK):
            oh = _onehot(b3_r[i, 0, :])
            cnt = cnt + jnp.sum(oh, axis=0)
        emb = pooled / jnp.maximum(cnt, 1.0)[:, None]
        emb_r[...] = emb
        log_r[...] = jnp.dot(emb, wp_r[...],
                             preferred_element_type=jnp.float32) + vecs_r[0, :]

    return pl.pallas_call(
        body,
        out_shape=[
            jax.ShapeDtypeStruct((G, DP), jnp.float32),
            jax.ShapeDtypeStruct((G, T), jnp.float32),
        ],
    )(ss, batch3, wp, vecs)


def _pad_cols(a, w):
    return jnp.pad(a, [(0, 0)] * (a.ndim - 1) + [(0, w - a.shape[-1])])


def _vec8(*rows):
    w = rows[0].shape[-1]
    out = jnp.zeros((8, w), jnp.float32)
    for i, r in enumerate(rows):
        out = out.at[i, :].set(r)
    return out


def kernel(x, edge_index, edge_attr, batch, atom_tables, bond_tables, eps,
           W1, b1, bnm_g, bnm_b, W2, b2, bn_g, bn_b, Wv1, bv1, bnv1_g,
           bnv1_b, Wv2, bv2, bnv2_g, bnv2_b, Wp, bp):
    f32 = jnp.float32
    i32 = jnp.int32

    atab = _pad_cols(atom_tables.astype(f32).reshape(9 * 128, D), DP)
    atlo, athi = atab[:, :DH], atab[:, DH:]

    xi = x.astype(i32) + 128 * jnp.arange(9, dtype=i32)[None, :]
    xi = jnp.pad(xi, ((0, NP - N), (0, 0)))
    xidx = xi.T.reshape(-1)

    cidx = jnp.arange(512, dtype=i32)
    bt = bond_tables.astype(f32)
    ctab = (bt[:, 0, cidx >> 6, :] + bt[:, 1, (cidx >> 3) & 7, :]
            + bt[:, 2, cidx & 7, :])
    ctab = _pad_cols(ctab, DP)
    ctlo, cthi = ctab[:, :, :DH], ctab[:, :, DH:]

    ea = edge_attr.astype(i32)
    code = ea[:, 0] * 64 + ea[:, 1] * 8 + ea[:, 2]
    srcp = jnp.pad(edge_index[0].astype(i32), (0, EP - E))
    dstp = jnp.pad(edge_index[1].astype(i32), (0, EP - E),
                   constant_values=N + 16)
    codep = jnp.pad(code, (0, EP - E))

    batch_p = jnp.pad(batch.astype(i32), (0, NP - N), constant_values=G)
    batch3 = batch_p.reshape(NBLK, 1, BN_ROWS)

    W1p = _pad_cols(jnp.pad(W1.astype(f32), ((0, 0), (0, DP - D), (0, 0))), HP)
    W2p = _pad_cols(jnp.pad(W2.astype(f32), ((0, 0), (0, HP - H2), (0, 0))), DP)
    Wv1p = _pad_cols(jnp.pad(Wv1.astype(f32), ((0, 0), (0, DP - D), (0, 0))), HP)
    Wv2p = _pad_cols(jnp.pad(Wv2.astype(f32), ((0, 0), (0, HP - H2), (0, 0))), DP)
    Wpp = jnp.pad(Wp.astype(f32), ((0, DP - D), (0, 0)))

    b1p = _pad_cols(b1.astype(f32), HP)
    b2p = _pad_cols(b2.astype(f32), DP)
    one = jnp.ones((HP,), f32)
    vecsA = [_vec8(b1p[l], one * (1.0 + eps[l])) for l in range(L)]
    gb1 = [_vec8(_pad_cols(bnm_g.astype(f32), HP)[l],
                 _pad_cols(bnm_b.astype(f32), HP)[l]) for l in range(L)]
    vecsB = [_vec8(b2p[l]) for l in range(L)]
    gb2 = [_vec8(_pad_cols(bn_g.astype(f32), DP)[l],
                 _pad_cols(bn_b.astype(f32), DP)[l]) for l in range(L)]
    vecsV1 = [_vec8(_pad_cols(bv1.astype(f32), HP)[l],
                    _pad_cols(bnv1_g.astype(f32), HP)[l],
                    _pad_cols(bnv1_b.astype(f32), HP)[l]) for l in range(L - 1)]
    vecsV2 = [_vec8(_pad_cols(bv2.astype(f32), DP)[l],
                    _pad_cols(bnv2_g.astype(f32), DP)[l],
                    _pad_cols(bnv2_b.astype(f32), DP)[l]) for l in range(L - 1)]
    vecsP = _vec8(bp.astype(f32))

    hin_lo, hin_hi = _sc_embed(atlo, athi, xidx)
    vn = jnp.zeros((G, DP), f32)
    s_list = []
    s5 = None
    for l in range(L):
        alo, ahi = _sc_edge(hin_lo, hin_hi, ctlo[l], cthi[l],
                            srcp, dstp, codep)
        z1, st1, s_l = _pass_a(hin_lo, hin_hi, alo, ahi, W1p[l], vecsA[l],
                               batch3)
        s_list.append(s_l)
        z2, st2 = _pass_b(z1, st1, gb1[l], W2p[l], vecsB[l])
        if l < L - 1:
            vn = _vn_mlp(s_l, vn, Wv1p[l], vecsV1[l], Wv2p[l], vecsV2[l])
            hin_lo, hin_hi = _pass_c(z2, st2, gb2[l], vn, batch3)
        else:
            s5 = _pass_c_last(z2, st2, gb2[l], batch3)

    ss = jnp.stack(s_list + [s5], axis=0)
    emb_p, logits = _final(ss, batch3, Wpp, vecsP)
    return emb_p[:, :D], logits


# speedup vs baseline: 1.8024x; 1.8024x over previous
import jax
import jax.numpy as jnp
from jax import lax
from jax.experimental import pallas as pl
from jax.experimental.pallas import tpu as pltpu
from jax.experimental.pallas import tpu_sc as plsc

N = 10000
NP = 10240
E = 160000
EP = 163840
D = 300
DP = 384
GW = 64
NG = 6
H2 = 600
HP = 768
G = 128
L = 5
T = 128
CC = 640
BN_ROWS = 1024
NBLK = NP // BN_ROWS
CH = 128
EDGES_PER_SUB = EP // 16
ECHUNKS = EDGES_PER_SUB // CH
ROWS_PER_SUB = NP // 16
RCHUNKS = ROWS_PER_SUB // CH

def _sc_params():
    return dict(
        mesh=plsc.VectorSubcoreMesh(core_axis_name="c", subcore_axis_name="s",
                                    num_cores=2, num_subcores=16),
        compiler_params=pltpu.CompilerParams(use_tc_tiling_on_sc=False),
    )


def _zero_rows(buf):
    z = jnp.zeros((16,), jnp.float32)

    def row(r, _):
        for j in range(GW // 16):
            buf[r, pl.ds(j * 16, 16)] = z
        return 0

    lax.fori_loop(0, CH, row, 0)


def _sc_embed(atgs, xidx):
    def body(a0, a1, a2, a3, a4, a5, xidx_r, o0, o1, o2, o3, o4, o5,
             idxv, gbuf, accbuf, sem):
        c = lax.axis_index("c")
        s = lax.axis_index("s")
        tabs = (a0, a1, a2, a3, a4, a5)
        outs = (o0, o1, o2, o3, o4, o5)

        def process(tab_ref, out_ref):
            def chunk(k, _):
                rowbase = (s * RCHUNKS + k) * CH
                for f in range(9):
                    pltpu.sync_copy(xidx_r.at[pl.ds(f * NP + rowbase, CH)], idxv)
                    pltpu.async_copy(tab_ref.at[idxv], gbuf, sem).wait()

                    if f == 0:
                        def rowcp(r, _):
                            for j in range(GW // 16):
                                sl = pl.ds(j * 16, 16)
                                accbuf[r, sl] = gbuf[r, sl]
                            return 0
                        lax.fori_loop(0, CH, rowcp, 0)
                    else:
                        def rowadd(r, _):
                            for j in range(GW // 16):
                                sl = pl.ds(j * 16, 16)
                                accbuf[r, sl] = accbuf[r, sl] + gbuf[r, sl]
                            return 0
                        lax.fori_loop(0, CH, rowadd, 0)
                pltpu.sync_copy(accbuf, out_ref.at[pl.ds(rowbase, CH)])
                return 0

            lax.fori_loop(0, RCHUNKS, chunk, 0)

        @pl.when(c == 0)
        def _():
            for g in range(3):
                process(tabs[g], outs[g])

        @pl.when(c == 1)
        def _():
            for g in range(3, 6):
                process(tabs[g], outs[g])

    f = pl.kernel(
        body,
        out_type=[jax.ShapeDtypeStruct((NP, GW), jnp.float32)
                  for _ in range(NG)],
        scratch_types=[
            pltpu.VMEM((CH,), jnp.int32),
            pltpu.VMEM((CH, GW), jnp.float32),
            pltpu.VMEM((CH, GW), jnp.float32),
            pltpu.SemaphoreType.DMA,
        ],
        **_sc_params(),
    )
    return list(f(*atgs, xidx))


def _sc_edge(ha, hb, cta, ctb, srcp, dstp, codep):
    def body(ha_r, hb_r, cta_r, ctb_r, src_r, dst_r, code_r,
             oa_r, ob_r, srcv, dstv, codev, hbuf, ebuf, tmpbuf,
             aggr_sh, sem1, sem2):
        c = lax.axis_index("c")
        s = lax.axis_index("s")

        _zero_rows(tmpbuf)
        for k in range(RCHUNKS):
            pltpu.sync_copy(tmpbuf, aggr_sh.at[pl.ds((s * RCHUNKS + k) * CH, CH)])
        plsc.subcore_barrier()

        def process(h_ref, ct_ref, out_ref):
            def chunk(k, _):
                base = s * EDGES_PER_SUB + k * CH
                pltpu.sync_copy(src_r.at[pl.ds(base, CH)], srcv)
                pltpu.sync_copy(code_r.at[pl.ds(base, CH)], codev)
                pltpu.sync_copy(dst_r.at[pl.ds(base, CH)], dstv)
                cp1 = pltpu.async_copy(h_ref.at[srcv], hbuf, sem1)
                cp2 = pltpu.async_copy(ct_ref.at[codev], ebuf, sem2)
                cp1.wait()
                cp2.wait()

                def row(r, _):
                    for j in range(GW // 16):
                        sl = pl.ds(j * 16, 16)
                        ebuf[r, sl] = jnp.maximum(hbuf[r, sl] + ebuf[r, sl], 0.0)
                    return 0

                lax.fori_loop(0, CH, row, 0)
                pltpu.sync_copy(ebuf, aggr_sh.at[dstv], add=True)
                return 0

            lax.fori_loop(0, ECHUNKS, chunk, 0)
            plsc.subcore_barrier()
            for k in range(RCHUNKS):
                r0 = (s * RCHUNKS + k) * CH
                pltpu.sync_copy(aggr_sh.at[pl.ds(r0, CH)], tmpbuf)
                pltpu.sync_copy(tmpbuf, out_ref.at[pl.ds(r0, CH)])

        @pl.when(c == 0)
        def _():
            process(ha_r, cta_r, oa_r)

        @pl.when(c == 1)
        def _():
            process(hb_r, ctb_r, ob_r)

    f = pl.kernel(
        body,
        out_type=[
            jax.ShapeDtypeStruct((NP, GW), jnp.float32),
            jax.ShapeDtypeStruct((NP, GW), jnp.float32),
        ],
        scratch_types=[
            pltpu.VMEM((CH,), jnp.int32),
            pltpu.VMEM((CH,), jnp.int32),
            pltpu.VMEM((CH,), jnp.int32),
            pltpu.VMEM((CH, GW), jnp.float32),
            pltpu.VMEM((CH, GW), jnp.float32),
            pltpu.VMEM((CH, GW), jnp.float32),
            pltpu.VMEM_SHARED((NP, GW), jnp.float32),
            pltpu.SemaphoreType.DMA,
            pltpu.SemaphoreType.DMA,
        ],
        **_sc_params(),
    )
    return f(ha, hb, cta, ctb, srcp, dstp, codep)


def _sc_edge_all(hgs, ctgs, srcp, dstp, codep):
    ags = [None] * NG
    for p in range(NG // 2):
        a, b = 2 * p, 2 * p + 1
        ags[a], ags[b] = _sc_edge(hgs[a], hgs[b], ctgs[a], ctgs[b],
                                  srcp, dstp, codep)
    return ags


def _onehot(b):
    return (b[:, None] == lax.broadcasted_iota(jnp.int32, (BN_ROWS, G), 1)
            ).astype(jnp.float32)


def _stats8(x, mask):
    xm = jnp.where(mask, x, 0.0)
    s0 = jnp.sum(xm, axis=0, keepdims=True)
    s1 = jnp.sum(xm * xm, axis=0, keepdims=True)
    return jnp.concatenate(
        [s0, s1, jnp.zeros((6, x.shape[1]), jnp.float32)], axis=0)


def _scale_shift(stats, g, b):
    m = stats[0, :] * (1.0 / N)
    v = stats[1, :] * (1.0 / N) - m * m
    scale = g * lax.rsqrt(v + 1e-5)
    return scale, b - m * scale


def _grp_spec():
    return pl.BlockSpec((BN_ROWS, GW), lambda i: (i, 0))


def _pass_a(hgs, ags, w1, vecs, batch3):
    def body(h0, h1, h2, h3, h4, h5, a0, a1, a2, a3, a4, a5,
             w1_r, vecs_r, b3_r, z1_r, st_r, s_r):
        i = pl.program_id(0)
        hin = jnp.concatenate([h[...] for h in (h0, h1, h2, h3, h4, h5)],
                              axis=1)
        aggr = jnp.concatenate([a[...] for a in (a0, a1, a2, a3, a4, a5)],
                               axis=1)
        z = hin * vecs_r[1, 0] + aggr
        z1 = jnp.dot(z, w1_r[...], preferred_element_type=jnp.float32)
        z1 = z1 + vecs_r[0, :]
        z1_r[...] = z1
        oh = _onehot(b3_r[0, 0, :])
        sacc = lax.dot_general(oh, hin, (((0,), (0,)), ((), ())),
                               preferred_element_type=jnp.float32)
        rowid = i * BN_ROWS + lax.broadcasted_iota(jnp.int32, (BN_ROWS, 1), 0)
        st = _stats8(z1, rowid < N)

        @pl.when(i == 0)
        def _():
            st_r[...] = st
            s_r[...] = sacc

        @pl.when(i > 0)
        def _():
            st_r[...] = st_r[...] + st
            s_r[...] = s_r[...] + sacc

    return pl.pallas_call(
        body,
        grid=(NBLK,),
        in_specs=[_grp_spec() for _ in range(2 * NG)] + [
            pl.BlockSpec((DP, HP), lambda i: (0, 0)),
            pl.BlockSpec((8, HP), lambda i: (0, 0)),
            pl.BlockSpec((1, 1, BN_ROWS), lambda i: (i, 0, 0)),
        ],
        out_specs=[
            pl.BlockSpec((BN_ROWS, HP), lambda i: (i, 0)),
            pl.BlockSpec((8, HP), lambda i: (0, 0)),
            pl.BlockSpec((G, DP), lambda i: (0, 0)),
        ],
        out_shape=[
            jax.ShapeDtypeStruct((NP, HP), jnp.float32),
            jax.ShapeDtypeStruct((8, HP), jnp.float32),
            jax.ShapeDtypeStruct((G, DP), jnp.float32),
        ],
    )(*hgs, *ags, w1, vecs, batch3)


def _pass_b(z1, st1, gb, w2, vecs):
    def body(z1_r, st_r, gb_r, w2_r, vecs_r, z2_r, st2_r):
        i = pl.program_id(0)
        scale, shift = _scale_shift(st_r[...], gb_r[0, :], gb_r[1, :])
        a = jnp.maximum(z1_r[...] * scale + shift, 0.0)
        z2 = jnp.dot(a, w2_r[...], preferred_element_type=jnp.float32)
        z2 = z2 + vecs_r[0, :]
        z2_r[...] = z2
        rowid = i * BN_ROWS + lax.broadcasted_iota(jnp.int32, (BN_ROWS, 1), 0)
        st = _stats8(z2, rowid < N)

        @pl.when(i == 0)
        def _():
            st2_r[...] = st

        @pl.when(i > 0)
        def _():
            st2_r[...] = st2_r[...] + st

    return pl.pallas_call(
        body,
        grid=(NBLK,),
        in_specs=[
            pl.BlockSpec((BN_ROWS, HP), lambda i: (i, 0)),
            pl.BlockSpec((8, HP), lambda i: (0, 0)),
            pl.BlockSpec((8, HP), lambda i: (0, 0)),
            pl.BlockSpec((HP, DP), lambda i: (0, 0)),
            pl.BlockSpec((8, DP), lambda i: (0, 0)),
        ],
        out_specs=[
            pl.BlockSpec((BN_ROWS, DP), lambda i: (i, 0)),
            pl.BlockSpec((8, DP), lambda i: (0, 0)),
        ],
        out_shape=[
            jax.ShapeDtypeStruct((NP, DP), jnp.float32),
            jax.ShapeDtypeStruct((8, DP), jnp.float32),
        ],
    )(z1, st1, gb, w2, vecs)


def _pass_c(z2, st2, gb, vn, batch3):
    def body(z2_r, st_r, gb_r, vn_r, b3_r, *outs):
        scale, shift = _scale_shift(st_r[...], gb_r[0, :], gb_r[1, :])
        h = jnp.maximum(z2_r[...] * scale + shift, 0.0)
        oh = _onehot(b3_r[0, 0, :])
        hin = h + jnp.dot(oh, vn_r[...], preferred_element_type=jnp.float32)
        for g in range(NG):
            outs[g][...] = hin[:, g * GW:(g + 1) * GW]

    return pl.pallas_call(
        body,
        grid=(NBLK,),
        in_specs=[
            pl.BlockSpec((BN_ROWS, DP), lambda i: (i, 0)),
            pl.BlockSpec((8, DP), lambda i: (0, 0)),
            pl.BlockSpec((8, DP), lambda i: (0, 0)),
            pl.BlockSpec((G, DP), lambda i: (0, 0)),
            pl.BlockSpec((1, 1, BN_ROWS), lambda i: (i, 0, 0)),
        ],
        out_specs=[_grp_spec() for _ in range(NG)],
        out_shape=[jax.ShapeDtypeStruct((NP, GW), jnp.float32)
                   for _ in range(NG)],
    )(z2, st2, gb, vn, batch3)


def _pass_c_last(z2, st2, gb, batch3):
    def body(z2_r, st_r, gb_r, b3_r, s5_r):
        i = pl.program_id(0)
        scale, shift = _scale_shift(st_r[...], gb_r[0, :], gb_r[1, :])
        h = z2_r[...] * scale + shift
        oh = _onehot(b3_r[0, 0, :])
        sacc = lax.dot_general(oh, h, (((0,), (0,)), ((), ())),
                               preferred_element_type=jnp.float32)

        @pl.when(i == 0)
        def _():
            s5_r[...] = sacc

        @pl.when(i > 0)
        def _():
            s5_r[...] = s5_r[...] + sacc

    return pl.pallas_call(
        body,
        grid=(NBLK,),
        in_specs=[
            pl.BlockSpec((BN_ROWS, DP), lambda i: (i, 0)),
            pl.BlockSpec((8, DP), lambda i: (0, 0)),
            pl.BlockSpec((8, DP), lambda i: (0, 0)),
            pl.BlockSpec((1, 1, BN_ROWS), lambda i: (i, 0, 0)),
        ],
        out_specs=[pl.BlockSpec((G, DP), lambda i: (0, 0))],
        out_shape=[jax.ShapeDtypeStruct((G, DP), jnp.float32)],
    )(z2, st2, gb, batch3)


def _vn_mlp(s_in, vn_in, w1, vecs1, w2, vecs2):
    def body(s_r, vn_r, w1_r, v1_r, w2_r, v2_r, out_r):
        svn = s_r[...] + vn_r[...]
        u = jnp.dot(svn, w1_r[...], preferred_element_type=jnp.float32)
        u = u + v1_r[0, :]
        m = jnp.mean(u, axis=0)
        v = jnp.mean(u * u, axis=0) - m * m
        u = jnp.maximum(v1_r[1, :] * (u - m) * lax.rsqrt(v + 1e-5)
                        + v1_r[2, :], 0.0)
        u2 = jnp.dot(u, w2_r[...], preferred_element_type=jnp.float32)
        u2 = u2 + v2_r[0, :]
        m2 = jnp.mean(u2, axis=0)
        vv2 = jnp.mean(u2 * u2, axis=0) - m2 * m2
        out_r[...] = jnp.maximum(
            v2_r[1, :] * (u2 - m2) * lax.rsqrt(vv2 + 1e-5) + v2_r[2, :], 0.0)

    return pl.pallas_call(
        body,
        out_shape=jax.ShapeDtypeStruct((G, DP), jnp.float32),
    )(s_in, vn_in, w1, vecs1, w2, vecs2)


def _final(ss, batch3, wp, vecs):
    def body(ss_r, b3_r, wp_r, vecs_r, emb_r, log_r):
        pooled = jnp.sum(ss_r[...], axis=0)
        cnt = jnp.zeros((G,), jnp.float32)
        for i in range(NBLK):
            oh = _onehot(b3_r[i, 0, :])
            cnt = cnt + jnp.sum(oh, axis=0)
        emb = pooled / jnp.maximum(cnt, 1.0)[:, None]
        emb_r[...] = emb
        log_r[...] = jnp.dot(emb, wp_r[...],
                             preferred_element_type=jnp.float32) + vecs_r[0, :]

    return pl.pallas_call(
        body,
        out_shape=[
            jax.ShapeDtypeStruct((G, DP), jnp.float32),
            jax.ShapeDtypeStruct((G, T), jnp.float32),
        ],
    )(ss, batch3, wp, vecs)


def _pad_cols(a, w):
    return jnp.pad(a, [(0, 0)] * (a.ndim - 1) + [(0, w - a.shape[-1])])


def _vec8(*rows):
    w = rows[0].shape[-1]
    out = jnp.zeros((8, w), jnp.float32)
    for i, r in enumerate(rows):
        out = out.at[i, :].set(r)
    return out


def _groups(a):
    return [a[..., g * GW:(g + 1) * GW] for g in range(NG)]


def kernel(x, edge_index, edge_attr, batch, atom_tables, bond_tables, eps,
           W1, b1, bnm_g, bnm_b, W2, b2, bn_g, bn_b, Wv1, bv1, bnv1_g,
           bnv1_b, Wv2, bv2, bnv2_g, bnv2_b, Wp, bp):
    f32 = jnp.float32
    i32 = jnp.int32

    atab = _pad_cols(atom_tables.astype(f32).reshape(9 * 128, D), DP)
    atgs = _groups(atab)

    xi = x.astype(i32) + 128 * jnp.arange(9, dtype=i32)[None, :]
    xi = jnp.pad(xi, ((0, NP - N), (0, 0)))
    xidx = xi.T.reshape(-1)

    cidx = jnp.arange(512, dtype=i32)
    bt = bond_tables.astype(f32)
    ctab = (bt[:, 0, cidx >> 6, :] + bt[:, 1, (cidx >> 3) & 7, :]
            + bt[:, 2, cidx & 7, :])
    ctab = _pad_cols(ctab, DP)
    ctab = jnp.concatenate(
        [ctab, jnp.full((L, CC - 512, DP), -1e30, f32)], axis=1)
    ctgs = [_groups(ctab[l]) for l in range(L)]

    ea = edge_attr.astype(i32)
    code = ea[:, 0] * 64 + ea[:, 1] * 8 + ea[:, 2]
    srcp = jnp.pad(edge_index[0].astype(i32), (0, EP - E))
    dstp = jnp.pad(edge_index[1].astype(i32), (0, EP - E))
    codep = jnp.pad(code, (0, EP - E), constant_values=512)

    batch_p = jnp.pad(batch.astype(i32), (0, NP - N), constant_values=G)
    batch3 = batch_p.reshape(NBLK, 1, BN_ROWS)

    W1p = _pad_cols(jnp.pad(W1.astype(f32), ((0, 0), (0, DP - D), (0, 0))), HP)
    W2p = _pad_cols(jnp.pad(W2.astype(f32), ((0, 0), (0, HP - H2), (0, 0))), DP)
    Wv1p = _pad_cols(jnp.pad(Wv1.astype(f32), ((0, 0), (0, DP - D), (0, 0))), HP)
    Wv2p = _pad_cols(jnp.pad(Wv2.astype(f32), ((0, 0), (0, HP - H2), (0, 0))), DP)
    Wpp = jnp.pad(Wp.astype(f32), ((0, DP - D), (0, 0)))

    b1p = _pad_cols(b1.astype(f32), HP)
    b2p = _pad_cols(b2.astype(f32), DP)
    one = jnp.ones((HP,), f32)
    vecsA = [_vec8(b1p[l], one * (1.0 + eps[l])) for l in range(L)]
    gb1 = [_vec8(_pad_cols(bnm_g.astype(f32), HP)[l],
                 _pad_cols(bnm_b.astype(f32), HP)[l]) for l in range(L)]
    vecsB = [_vec8(b2p[l]) for l in range(L)]
    gb2 = [_vec8(_pad_cols(bn_g.astype(f32), DP)[l],
                 _pad_cols(bn_b.astype(f32), DP)[l]) for l in range(L)]
    vecsV1 = [_vec8(_pad_cols(bv1.astype(f32), HP)[l],
                    _pad_cols(bnv1_g.astype(f32), HP)[l],
                    _pad_cols(bnv1_b.astype(f32), HP)[l]) for l in range(L - 1)]
    vecsV2 = [_vec8(_pad_cols(bv2.astype(f32), DP)[l],
                    _pad_cols(bnv2_g.astype(f32), DP)[l],
                    _pad_cols(bnv2_b.astype(f32), DP)[l]) for l in range(L - 1)]
    vecsP = _vec8(bp.astype(f32))

    hgs = _sc_embed(atgs, xidx)
    vn = jnp.zeros((G, DP), f32)
    s_list = []
    s5 = None
    for l in range(L):
        ags = _sc_edge_all(hgs, ctgs[l], srcp, dstp, codep)
        z1, st1, s_l = _pass_a(hgs, ags, W1p[l], vecsA[l], batch3)
        s_list.append(s_l)
        z2, st2 = _pass_b(z1, st1, gb1[l], W2p[l], vecsB[l])
        if l < L - 1:
            vn = _vn_mlp(s_l, vn, Wv1p[l], vecsV1[l], Wv2p[l], vecsV2[l])
            hgs = _pass_c(z2, st2, gb2[l], vn, batch3)
        else:
            (s5,) = _pass_c_last(z2, st2, gb2[l], batch3)

    ss = jnp.stack(s_list + [s5], axis=0)
    emb_p, logits = _final(ss, batch3, Wpp, vecsP)
    return emb_p[:, :D], logits


# speedup vs baseline: 2.3568x; 1.3076x over previous
import jax
import jax.numpy as jnp
from jax import lax
from jax.experimental import pallas as pl
from jax.experimental.pallas import tpu as pltpu
from jax.experimental.pallas import tpu_sc as plsc

N = 10000
NP = 10240
E = 160000
EP = 163840
D = 300
DP = 384
GW = 64
NG = 6
H2 = 600
HP = 768
G = 128
L = 5
T = 128
CC = 640
BN_ROWS = 1024
NBLK = NP // BN_ROWS
CH = 128
EDGES_PER_SUB = EP // 16
ECHUNKS = EDGES_PER_SUB // CH
ROWS_PER_SUB = NP // 16
RCHUNKS = ROWS_PER_SUB // CH

def _sc_params():
    return dict(
        mesh=plsc.VectorSubcoreMesh(core_axis_name="c", subcore_axis_name="s",
                                    num_cores=2, num_subcores=16),
        compiler_params=pltpu.CompilerParams(use_tc_tiling_on_sc=False),
    )


def _zero_rows(buf):
    z = jnp.zeros((16,), jnp.float32)

    def row(r, _):
        for j in range(GW // 16):
            buf[r, pl.ds(j * 16, 16)] = z
        return 0

    lax.fori_loop(0, CH, row, 0)


def _sc_embed(at6, xidxT):
    def body(at_r, xidx_r, out_r, idxall, g0, g1, accbuf, sem0, sem1):
        c = lax.axis_index("c")
        s = lax.axis_index("s")
        gb = (g0, g1)
        sems = (sem0, sem1)

        pltpu.sync_copy(xidx_r.at[s], idxall)

        def per_group(gi, _):
            g = c * 3 + gi
            tab_ref = at_r.at[g]

            def per_chunk(k, _):
                rowbase = (s * RCHUNKS + k) * CH

                def idxk(f):
                    return idxall.at[f, pl.ds(k * CH, CH)]

                pltpu.async_copy(tab_ref.at[idxk(0)], gb[0], sems[0])
                for f in range(9):
                    p = f % 2
                    if f < 8:
                        q = 1 - p
                        pltpu.async_copy(tab_ref.at[idxk(f + 1)], gb[q],
                                         sems[q])
                    pltpu.make_async_copy(tab_ref.at[idxk(f)], gb[p],
                                          sems[p]).wait()
                    src = gb[p]
                    if f == 0:
                        def rowcp(r, _):
                            for rr in range(2):
                                for j in range(GW // 16):
                                    sl = pl.ds(j * 16, 16)
                                    accbuf[2 * r + rr, sl] = src[2 * r + rr, sl]
                            return 0
                        lax.fori_loop(0, CH // 2, rowcp, 0)
                    else:
                        def rowadd(r, _):
                            for rr in range(2):
                                for j in range(GW // 16):
                                    sl = pl.ds(j * 16, 16)
                                    accbuf[2 * r + rr, sl] = (
                                        accbuf[2 * r + rr, sl]
                                        + src[2 * r + rr, sl])
                            return 0
                        lax.fori_loop(0, CH // 2, rowadd, 0)
                pltpu.sync_copy(accbuf, out_r.at[g, pl.ds(rowbase, CH)])
                return 0

            lax.fori_loop(0, RCHUNKS, per_chunk, 0)
            return 0

        lax.fori_loop(0, 3, per_group, 0)

    f = pl.kernel(
        body,
        out_type=jax.ShapeDtypeStruct((NG, NP, GW), jnp.float32),
        scratch_types=[
            pltpu.VMEM((9, ROWS_PER_SUB), jnp.int32),
            pltpu.VMEM((CH, GW), jnp.float32),
            pltpu.VMEM((CH, GW), jnp.float32),
            pltpu.VMEM((CH, GW), jnp.float32),
            pltpu.SemaphoreType.DMA,
            pltpu.SemaphoreType.DMA,
        ],
        **_sc_params(),
    )
    return f(at6, xidxT)


def _sc_edge_all(h6, ct6, srcT, dstT, codeT):
    def body(h_r, ct_r, src_r, dst_r, code_r, out_r,
             srcall, dstall, codeall,
             hb0, hb1, eb0, eb1, sb0, sb1, tmpbuf, aggr_sh,
             semh0, semh1, seme0, seme1, sems0, sems1):
        c = lax.axis_index("c")
        s = lax.axis_index("s")
        hb = (hb0, hb1)
        eb = (eb0, eb1)
        sb = (sb0, sb1)
        semh = (semh0, semh1)
        seme = (seme0, seme1)
        sems = (sems0, sems1)

        pltpu.sync_copy(src_r.at[s], srcall)
        pltpu.sync_copy(dst_r.at[s], dstall)
        pltpu.sync_copy(code_r.at[s], codeall)
        _zero_rows(tmpbuf)

        def per_group(gi, _):
            g = c * 3 + gi
            h_ref = h_r.at[g]
            ct_ref = ct_r.at[g]
            for k in range(RCHUNKS):
                pltpu.sync_copy(tmpbuf,
                                aggr_sh.at[pl.ds((s * RCHUNKS + k) * CH, CH)])
            plsc.subcore_barrier()

            def loop(k, _):
                p = 0
                cp1 = pltpu.async_copy(h_ref.at[srcall.at[k]], hb[p], semh[p])
                cp2 = pltpu.async_copy(ct_ref.at[codeall.at[k]], eb[p],
                                       seme[p])
                cp1.wait()
                cp2.wait()
                hp, ep, sp = hb[p], eb[p], sb[p]

                def row(r, _):
                    for rr in range(2):
                        for j in range(GW // 16):
                            sl = pl.ds(j * 16, 16)
                            sp[2 * r + rr, sl] = jnp.maximum(
                                hp[2 * r + rr, sl] + ep[2 * r + rr, sl],
                                0.0)
                    return 0

                lax.fori_loop(0, CH // 2, row, 0)
                pltpu.sync_copy(sp, aggr_sh.at[dstall.at[k]], add=True)
                return 0

            lax.fori_loop(0, ECHUNKS, loop, 0)
            plsc.subcore_barrier()
            for k in range(RCHUNKS):
                r0 = (s * RCHUNKS + k) * CH
                pltpu.sync_copy(aggr_sh.at[pl.ds(r0, CH)], hb[0])
                pltpu.sync_copy(hb[0], out_r.at[g, pl.ds(r0, CH)])
            return 0

        lax.fori_loop(0, 3, per_group, 0)

    f = pl.kernel(
        body,
        out_type=jax.ShapeDtypeStruct((NG, NP, GW), jnp.float32),
        scratch_types=[
            pltpu.VMEM((ECHUNKS, CH), jnp.int32),
            pltpu.VMEM((ECHUNKS, CH), jnp.int32),
            pltpu.VMEM((ECHUNKS, CH), jnp.int32),
            pltpu.VMEM((CH, GW), jnp.float32),
            pltpu.VMEM((CH, GW), jnp.float32),
            pltpu.VMEM((CH, GW), jnp.float32),
            pltpu.VMEM((CH, GW), jnp.float32),
            pltpu.VMEM((CH, GW), jnp.float32),
            pltpu.VMEM((CH, GW), jnp.float32),
            pltpu.VMEM((CH, GW), jnp.float32),
            pltpu.VMEM_SHARED((NP, GW), jnp.float32),
            pltpu.SemaphoreType.DMA,
            pltpu.SemaphoreType.DMA,
            pltpu.SemaphoreType.DMA,
            pltpu.SemaphoreType.DMA,
            pltpu.SemaphoreType.DMA,
            pltpu.SemaphoreType.DMA,
        ],
        **_sc_params(),
    )
    return f(h6, ct6, srcT, dstT, codeT)


def _onehot(b):
    return (b[:, None] == lax.broadcasted_iota(jnp.int32, (BN_ROWS, G), 1)
            ).astype(jnp.float32)


def _stats8(x, mask):
    xm = jnp.where(mask, x, 0.0)
    s0 = jnp.sum(xm, axis=0, keepdims=True)
    s1 = jnp.sum(xm * xm, axis=0, keepdims=True)
    return jnp.concatenate(
        [s0, s1, jnp.zeros((6, x.shape[1]), jnp.float32)], axis=0)


def _scale_shift(stats, g, b):
    m = stats[0, :] * (1.0 / N)
    v = stats[1, :] * (1.0 / N) - m * m
    scale = g * lax.rsqrt(v + 1e-5)
    return scale, b - m * scale


def _grp_spec():
    return pl.BlockSpec((BN_ROWS, GW), lambda i: (i, 0))


def _g6_spec(g):
    return pl.BlockSpec((1, BN_ROWS, GW), lambda i, g=g: (g, i, 0))


def _pass_a(h6, a6, w1, vecs, batch3):
    def body(h0, h1, h2, h3, h4, h5, a0, a1, a2, a3, a4, a5,
             w1_r, vecs_r, b3_r, z1_r, st_r, s_r):
        i = pl.program_id(0)
        hin = jnp.concatenate([h[0] for h in (h0, h1, h2, h3, h4, h5)],
                              axis=1)
        aggr = jnp.concatenate([a[0] for a in (a0, a1, a2, a3, a4, a5)],
                               axis=1)
        z = hin * vecs_r[1, 0] + aggr
        z1 = jnp.dot(z, w1_r[...], preferred_element_type=jnp.float32)
        z1 = z1 + vecs_r[0, :]
        z1_r[...] = z1
        oh = _onehot(b3_r[0, 0, :])
        sacc = lax.dot_general(oh, hin, (((0,), (0,)), ((), ())),
                               preferred_element_type=jnp.float32)
        rowid = i * BN_ROWS + lax.broadcasted_iota(jnp.int32, (BN_ROWS, 1), 0)
        st = _stats8(z1, rowid < N)

        @pl.when(i == 0)
        def _():
            st_r[...] = st
            s_r[...] = sacc

        @pl.when(i > 0)
        def _():
            st_r[...] = st_r[...] + st
            s_r[...] = s_r[...] + sacc

    return pl.pallas_call(
        body,
        grid=(NBLK,),
        in_specs=[_g6_spec(g) for g in range(NG)] * 2 + [
            pl.BlockSpec((DP, HP), lambda i: (0, 0)),
            pl.BlockSpec((8, HP), lambda i: (0, 0)),
            pl.BlockSpec((1, 1, BN_ROWS), lambda i: (i, 0, 0)),
        ],
        out_specs=[
            pl.BlockSpec((BN_ROWS, HP), lambda i: (i, 0)),
            pl.BlockSpec((8, HP), lambda i: (0, 0)),
            pl.BlockSpec((G, DP), lambda i: (0, 0)),
        ],
        out_shape=[
            jax.ShapeDtypeStruct((NP, HP), jnp.float32),
            jax.ShapeDtypeStruct((8, HP), jnp.float32),
            jax.ShapeDtypeStruct((G, DP), jnp.float32),
        ],
    )(*([h6] * NG), *([a6] * NG), w1, vecs, batch3)


def _pass_b(z1, st1, gb, w2, vecs):
    def body(z1_r, st_r, gb_r, w2_r, vecs_r, z2_r, st2_r):
        i = pl.program_id(0)
        scale, shift = _scale_shift(st_r[...], gb_r[0, :], gb_r[1, :])
        a = jnp.maximum(z1_r[...] * scale + shift, 0.0)
        z2 = jnp.dot(a, w2_r[...], preferred_element_type=jnp.float32)
        z2 = z2 + vecs_r[0, :]
        z2_r[...] = z2
        rowid = i * BN_ROWS + lax.broadcasted_iota(jnp.int32, (BN_ROWS, 1), 0)
        st = _stats8(z2, rowid < N)

        @pl.when(i == 0)
        def _():
            st2_r[...] = st

        @pl.when(i > 0)
        def _():
            st2_r[...] = st2_r[...] + st

    return pl.pallas_call(
        body,
        grid=(NBLK,),
        in_specs=[
            pl.BlockSpec((BN_ROWS, HP), lambda i: (i, 0)),
            pl.BlockSpec((8, HP), lambda i: (0, 0)),
            pl.BlockSpec((8, HP), lambda i: (0, 0)),
            pl.BlockSpec((HP, DP), lambda i: (0, 0)),
            pl.BlockSpec((8, DP), lambda i: (0, 0)),
        ],
        out_specs=[
            pl.BlockSpec((BN_ROWS, DP), lambda i: (i, 0)),
            pl.BlockSpec((8, DP), lambda i: (0, 0)),
        ],
        out_shape=[
            jax.ShapeDtypeStruct((NP, DP), jnp.float32),
            jax.ShapeDtypeStruct((8, DP), jnp.float32),
        ],
    )(z1, st1, gb, w2, vecs)


def _pass_c(z2, st2, gb, vn, batch3):
    def body(z2_r, st_r, gb_r, vn_r, b3_r, *outs):
        scale, shift = _scale_shift(st_r[...], gb_r[0, :], gb_r[1, :])
        h = jnp.maximum(z2_r[...] * scale + shift, 0.0)
        oh = _onehot(b3_r[0, 0, :])
        hin = h + jnp.dot(oh, vn_r[...], preferred_element_type=jnp.float32)
        for g in range(NG):
            outs[g][...] = hin[:, g * GW:(g + 1) * GW]

    return pl.pallas_call(
        body,
        grid=(NBLK,),
        in_specs=[
            pl.BlockSpec((BN_ROWS, DP), lambda i: (i, 0)),
            pl.BlockSpec((8, DP), lambda i: (0, 0)),
            pl.BlockSpec((8, DP), lambda i: (0, 0)),
            pl.BlockSpec((G, DP), lambda i: (0, 0)),
            pl.BlockSpec((1, 1, BN_ROWS), lambda i: (i, 0, 0)),
        ],
        out_specs=[_grp_spec() for _ in range(NG)],
        out_shape=[jax.ShapeDtypeStruct((NP, GW), jnp.float32)
                   for _ in range(NG)],
    )(z2, st2, gb, vn, batch3)


def _pass_c_last(z2, st2, gb, batch3):
    def body(z2_r, st_r, gb_r, b3_r, s5_r):
        i = pl.program_id(0)
        scale, shift = _scale_shift(st_r[...], gb_r[0, :], gb_r[1, :])
        h = z2_r[...] * scale + shift
        oh = _onehot(b3_r[0, 0, :])
        sacc = lax.dot_general(oh, h, (((0,), (0,)), ((), ())),
                               preferred_element_type=jnp.float32)

        @pl.when(i == 0)
        def _():
            s5_r[...] = sacc

        @pl.when(i > 0)
        def _():
            s5_r[...] = s5_r[...] + sacc

    return pl.pallas_call(
        body,
        grid=(NBLK,),
        in_specs=[
            pl.BlockSpec((BN_ROWS, DP), lambda i: (i, 0)),
            pl.BlockSpec((8, DP), lambda i: (0, 0)),
            pl.BlockSpec((8, DP), lambda i: (0, 0)),
            pl.BlockSpec((1, 1, BN_ROWS), lambda i: (i, 0, 0)),
        ],
        out_specs=[pl.BlockSpec((G, DP), lambda i: (0, 0))],
        out_shape=[jax.ShapeDtypeStruct((G, DP), jnp.float32)],
    )(z2, st2, gb, batch3)


def _vn_mlp(s_in, vn_in, w1, vecs1, w2, vecs2):
    def body(s_r, vn_r, w1_r, v1_r, w2_r, v2_r, out_r):
        svn = s_r[...] + vn_r[...]
        u = jnp.dot(svn, w1_r[...], preferred_element_type=jnp.float32)
        u = u + v1_r[0, :]
        m = jnp.mean(u, axis=0)
        v = jnp.mean(u * u, axis=0) - m * m
        u = jnp.maximum(v1_r[1, :] * (u - m) * lax.rsqrt(v + 1e-5)
                        + v1_r[2, :], 0.0)
        u2 = jnp.dot(u, w2_r[...], preferred_element_type=jnp.float32)
        u2 = u2 + v2_r[0, :]
        m2 = jnp.mean(u2, axis=0)
        vv2 = jnp.mean(u2 * u2, axis=0) - m2 * m2
        out_r[...] = jnp.maximum(
            v2_r[1, :] * (u2 - m2) * lax.rsqrt(vv2 + 1e-5) + v2_r[2, :], 0.0)

    return pl.pallas_call(
        body,
        out_shape=jax.ShapeDtypeStruct((G, DP), jnp.float32),
    )(s_in, vn_in, w1, vecs1, w2, vecs2)


def _final(ss, batch3, wp, vecs):
    def body(ss_r, b3_r, wp_r, vecs_r, emb_r, log_r):
        pooled = jnp.sum(ss_r[...], axis=0)
        cnt = jnp.zeros((G,), jnp.float32)
        for i in range(NBLK):
            oh = _onehot(b3_r[i, 0, :])
            cnt = cnt + jnp.sum(oh, axis=0)
        emb = pooled / jnp.maximum(cnt, 1.0)[:, None]
        emb_r[...] = emb
        log_r[...] = jnp.dot(emb, wp_r[...],
                             preferred_element_type=jnp.float32) + vecs_r[0, :]

    return pl.pallas_call(
        body,
        out_shape=[
            jax.ShapeDtypeStruct((G, DP), jnp.float32),
            jax.ShapeDtypeStruct((G, T), jnp.float32),
        ],
    )(ss, batch3, wp, vecs)


def _pad_cols(a, w):
    return jnp.pad(a, [(0, 0)] * (a.ndim - 1) + [(0, w - a.shape[-1])])


def _vec8(*rows):
    w = rows[0].shape[-1]
    out = jnp.zeros((8, w), jnp.float32)
    for i, r in enumerate(rows):
        out = out.at[i, :].set(r)
    return out


def _groups(a):
    return [a[..., g * GW:(g + 1) * GW] for g in range(NG)]


def kernel(x, edge_index, edge_attr, batch, atom_tables, bond_tables, eps,
           W1, b1, bnm_g, bnm_b, W2, b2, bn_g, bn_b, Wv1, bv1, bnv1_g,
           bnv1_b, Wv2, bv2, bnv2_g, bnv2_b, Wp, bp):
    f32 = jnp.float32
    i32 = jnp.int32

    atab = _pad_cols(atom_tables.astype(f32).reshape(9 * 128, D), DP)
    at6 = jnp.stack(_groups(atab), axis=0)

    xi = x.astype(i32) + 128 * jnp.arange(9, dtype=i32)[None, :]
    xi = jnp.pad(xi, ((0, NP - N), (0, 0)))
    xidxT = jnp.transpose(xi.T.reshape(9, 16, ROWS_PER_SUB), (1, 0, 2))

    cidx = jnp.arange(512, dtype=i32)
    bt = bond_tables.astype(f32)
    ctab = (bt[:, 0, cidx >> 6, :] + bt[:, 1, (cidx >> 3) & 7, :]
            + bt[:, 2, cidx & 7, :])
    ctab = _pad_cols(ctab, DP)
    ctab = jnp.concatenate(
        [ctab, jnp.full((L, CC - 512, DP), -1e30, f32)], axis=1)
    ct6 = [jnp.stack(_groups(ctab[l]), axis=0) for l in range(L)]

    ea = edge_attr.astype(i32)
    code = ea[:, 0] * 64 + ea[:, 1] * 8 + ea[:, 2]
    srcT = jnp.pad(edge_index[0].astype(i32),
                   (0, EP - E)).reshape(16, ECHUNKS, CH)
    dstT = jnp.pad(edge_index[1].astype(i32),
                   (0, EP - E)).reshape(16, ECHUNKS, CH)
    codeT = jnp.pad(code, (0, EP - E),
                    constant_values=512).reshape(16, ECHUNKS, CH)

    batch_p = jnp.pad(batch.astype(i32), (0, NP - N), constant_values=G)
    batch3 = batch_p.reshape(NBLK, 1, BN_ROWS)

    W1p = _pad_cols(jnp.pad(W1.astype(f32), ((0, 0), (0, DP - D), (0, 0))), HP)
    W2p = _pad_cols(jnp.pad(W2.astype(f32), ((0, 0), (0, HP - H2), (0, 0))), DP)
    Wv1p = _pad_cols(jnp.pad(Wv1.astype(f32), ((0, 0), (0, DP - D), (0, 0))), HP)
    Wv2p = _pad_cols(jnp.pad(Wv2.astype(f32), ((0, 0), (0, HP - H2), (0, 0))), DP)
    Wpp = jnp.pad(Wp.astype(f32), ((0, DP - D), (0, 0)))

    b1p = _pad_cols(b1.astype(f32), HP)
    b2p = _pad_cols(b2.astype(f32), DP)
    one = jnp.ones((HP,), f32)
    vecsA = [_vec8(b1p[l], one * (1.0 + eps[l])) for l in range(L)]
    gb1 = [_vec8(_pad_cols(bnm_g.astype(f32), HP)[l],
                 _pad_cols(bnm_b.astype(f32), HP)[l]) for l in range(L)]
    vecsB = [_vec8(b2p[l]) for l in range(L)]
    gb2 = [_vec8(_pad_cols(bn_g.astype(f32), DP)[l],
                 _pad_cols(bn_b.astype(f32), DP)[l]) for l in range(L)]
    vecsV1 = [_vec8(_pad_cols(bv1.astype(f32), HP)[l],
                    _pad_cols(bnv1_g.astype(f32), HP)[l],
                    _pad_cols(bnv1_b.astype(f32), HP)[l]) for l in range(L - 1)]
    vecsV2 = [_vec8(_pad_cols(bv2.astype(f32), DP)[l],
                    _pad_cols(bnv2_g.astype(f32), DP)[l],
                    _pad_cols(bnv2_b.astype(f32), DP)[l]) for l in range(L - 1)]
    vecsP = _vec8(bp.astype(f32))

    h6 = _sc_embed(at6, xidxT)
    vn = jnp.zeros((G, DP), f32)
    s_list = []
    s5 = None
    for l in range(L):
        a6 = _sc_edge_all(h6, ct6[l], srcT, dstT, codeT)
        z1, st1, s_l = _pass_a(h6, a6, W1p[l], vecsA[l], batch3)
        s_list.append(s_l)
        z2, st2 = _pass_b(z1, st1, gb1[l], W2p[l], vecsB[l])
        if l < L - 1:
            vn = _vn_mlp(s_l, vn, Wv1p[l], vecsV1[l], Wv2p[l], vecsV2[l])
            h6 = jnp.stack(_pass_c(z2, st2, gb2[l], vn, batch3), axis=0)
        else:
            (s5,) = _pass_c_last(z2, st2, gb2[l], batch3)

    ss = jnp.stack(s_list + [s5], axis=0)
    emb_p, logits = _final(ss, batch3, Wpp, vecsP)
    return emb_p[:, :D], logits


# speedup vs baseline: 2.6865x; 1.1399x over previous
import jax
import jax.numpy as jnp
from jax import lax
from jax.experimental import pallas as pl
from jax.experimental.pallas import tpu as pltpu
from jax.experimental.pallas import tpu_sc as plsc

N = 10000
NP = 10240
E = 160000
EP = 163840
D = 300
DP = 384
GW = 64
NG = 6
H2 = 600
HP = 768
G = 128
L = 5
T = 128
CC = 640
BN_ROWS = 1024
NBLK = NP // BN_ROWS
CH = 128
EDGES_PER_SUB = EP // 16
ECHUNKS = EDGES_PER_SUB // CH
ROWS_PER_SUB = NP // 16
RCHUNKS = ROWS_PER_SUB // CH

def _sc_params():
    return dict(
        mesh=plsc.VectorSubcoreMesh(core_axis_name="c", subcore_axis_name="s",
                                    num_cores=2, num_subcores=16),
        compiler_params=pltpu.CompilerParams(use_tc_tiling_on_sc=False),
    )


def _zero_rows(buf):
    z = jnp.zeros((16,), jnp.float32)

    def row(r, _):
        for j in range(GW // 16):
            buf[r, pl.ds(j * 16, 16)] = z
        return 0

    lax.fori_loop(0, CH, row, 0)


def _sc_embed(at6, xidxT):
    def body(at_r, xidx_r, out_r, idxall, g0, g1, accbuf, sem0, sem1):
        c = lax.axis_index("c")
        s = lax.axis_index("s")
        gb = (g0, g1)
        sems = (sem0, sem1)

        pltpu.sync_copy(xidx_r.at[s], idxall)

        def per_group(gi, _):
            g = c * 3 + gi
            tab_ref = at_r.at[g]

            def per_chunk(k, _):
                rowbase = (s * RCHUNKS + k) * CH

                def idxk(f):
                    return idxall.at[f, pl.ds(k * CH, CH)]

                pltpu.async_copy(tab_ref.at[idxk(0)], gb[0], sems[0])
                for f in range(9):
                    p = f % 2
                    if f < 8:
                        q = 1 - p
                        pltpu.async_copy(tab_ref.at[idxk(f + 1)], gb[q],
                                         sems[q])
                    pltpu.make_async_copy(tab_ref.at[idxk(f)], gb[p],
                                          sems[p]).wait()
                    src = gb[p]
                    if f == 0:
                        def rowcp(r, _):
                            for rr in range(2):
                                for j in range(GW // 16):
                                    sl = pl.ds(j * 16, 16)
                                    accbuf[2 * r + rr, sl] = src[2 * r + rr, sl]
                            return 0
                        lax.fori_loop(0, CH // 2, rowcp, 0)
                    else:
                        def rowadd(r, _):
                            for rr in range(2):
                                for j in range(GW // 16):
                                    sl = pl.ds(j * 16, 16)
                                    accbuf[2 * r + rr, sl] = (
                                        accbuf[2 * r + rr, sl]
                                        + src[2 * r + rr, sl])
                            return 0
                        lax.fori_loop(0, CH // 2, rowadd, 0)
                pltpu.sync_copy(accbuf, out_r.at[g, pl.ds(rowbase, CH)])
                return 0

            lax.fori_loop(0, RCHUNKS, per_chunk, 0)
            return 0

        lax.fori_loop(0, 3, per_group, 0)

    f = pl.kernel(
        body,
        out_type=jax.ShapeDtypeStruct((NG, NP, GW), jnp.float32),
        scratch_types=[
            pltpu.VMEM((9, ROWS_PER_SUB), jnp.int32),
            pltpu.VMEM((CH, GW), jnp.float32),
            pltpu.VMEM((CH, GW), jnp.float32),
            pltpu.VMEM((CH, GW), jnp.float32),
            pltpu.SemaphoreType.DMA,
            pltpu.SemaphoreType.DMA,
        ],
        **_sc_params(),
    )
    return f(at6, xidxT)


def _sc_edge_all(h6, ct6, srcT, dstT, codeT):
    def body(h_r, ct_r, src_r, dst_r, code_r, out_r,
             srcall, dstall, codeall,
             hb0, hb1, eb0, eb1, sb0, sb1, tmpbuf, aggr_sh,
             semh0, semh1, seme0, seme1, sems0, sems1):
        c = lax.axis_index("c")
        s = lax.axis_index("s")
        hb = (hb0, hb1)
        eb = (eb0, eb1)
        sb = (sb0, sb1)
        semh = (semh0, semh1)
        seme = (seme0, seme1)
        sems = (sems0, sems1)

        pltpu.sync_copy(src_r.at[s], srcall)
        pltpu.sync_copy(dst_r.at[s], dstall)
        pltpu.sync_copy(code_r.at[s], codeall)
        _zero_rows(tmpbuf)

        def per_group(gi, _):
            g = c * 3 + gi
            h_ref = h_r.at[g]
            ct_ref = ct_r.at[g]
            for k in range(RCHUNKS):
                pltpu.sync_copy(tmpbuf,
                                aggr_sh.at[pl.ds((s * RCHUNKS + k) * CH, CH)])
            plsc.subcore_barrier()

            def fire(k, p):
                pltpu.async_copy(h_ref.at[srcall.at[k]], hb[p], semh[p])
                pltpu.async_copy(ct_ref.at[codeall.at[k]], eb[p], seme[p])

            fire(0, 0)

            def loop(k2, _):
                for p in (0, 1):
                    k = k2 * 2 + p
                    nk = k + 1

                    @pl.when(nk < ECHUNKS)
                    def _():
                        q = 1 - p
                        pltpu.async_copy(h_ref.at[srcall.at[nk]], hb[q],
                                         semh[q])
                        pltpu.async_copy(ct_ref.at[codeall.at[nk]], eb[q],
                                         seme[q])

                    pltpu.make_async_copy(h_ref.at[srcall.at[k]], hb[p],
                                          semh[p]).wait()
                    pltpu.make_async_copy(ct_ref.at[codeall.at[k]], eb[p],
                                          seme[p]).wait()
                    hp, ep, sp = hb[p], eb[p], sb[p]

                    def row(r, _):
                        for rr in range(2):
                            for j in range(GW // 16):
                                sl = pl.ds(j * 16, 16)
                                sp[2 * r + rr, sl] = jnp.maximum(
                                    hp[2 * r + rr, sl] + ep[2 * r + rr, sl],
                                    0.0)
                        return 0

                    lax.fori_loop(0, CH // 2, row, 0)
                    pltpu.sync_copy(sp, aggr_sh.at[dstall.at[k]], add=True)
                return 0

            lax.fori_loop(0, ECHUNKS // 2, loop, 0)
            plsc.subcore_barrier()
            for k in range(RCHUNKS):
                r0 = (s * RCHUNKS + k) * CH
                pltpu.sync_copy(aggr_sh.at[pl.ds(r0, CH)], hb[0])
                pltpu.sync_copy(hb[0], out_r.at[g, pl.ds(r0, CH)])
            return 0

        lax.fori_loop(0, 3, per_group, 0)

    f = pl.kernel(
        body,
        out_type=jax.ShapeDtypeStruct((NG, NP, GW), jnp.float32),
        scratch_types=[
            pltpu.VMEM((ECHUNKS, CH), jnp.int32),
            pltpu.VMEM((ECHUNKS, CH), jnp.int32),
            pltpu.VMEM((ECHUNKS, CH), jnp.int32),
            pltpu.VMEM((CH, GW), jnp.float32),
            pltpu.VMEM((CH, GW), jnp.float32),
            pltpu.VMEM((CH, GW), jnp.float32),
            pltpu.VMEM((CH, GW), jnp.float32),
            pltpu.VMEM((CH, GW), jnp.float32),
            pltpu.VMEM((CH, GW), jnp.float32),
            pltpu.VMEM((CH, GW), jnp.float32),
            pltpu.VMEM_SHARED((NP, GW), jnp.float32),
            pltpu.SemaphoreType.DMA,
            pltpu.SemaphoreType.DMA,
            pltpu.SemaphoreType.DMA,
            pltpu.SemaphoreType.DMA,
            pltpu.SemaphoreType.DMA,
            pltpu.SemaphoreType.DMA,
        ],
        **_sc_params(),
    )
    return f(h6, ct6, srcT, dstT, codeT)


def _onehot(b):
    return (b[:, None] == lax.broadcasted_iota(jnp.int32, (BN_ROWS, G), 1)
            ).astype(jnp.float32)


def _stats8(x, mask):
    xm = jnp.where(mask, x, 0.0)
    s0 = jnp.sum(xm, axis=0, keepdims=True)
    s1 = jnp.sum(xm * xm, axis=0, keepdims=True)
    return jnp.concatenate(
        [s0, s1, jnp.zeros((6, x.shape[1]), jnp.float32)], axis=0)


def _scale_shift(stats, g, b):
    m = stats[0, :] * (1.0 / N)
    v = stats[1, :] * (1.0 / N) - m * m
    scale = g * lax.rsqrt(v + 1e-5)
    return scale, b - m * scale


def _grp_spec():
    return pl.BlockSpec((BN_ROWS, GW), lambda i: (i, 0))


def _g6_spec(g):
    return pl.BlockSpec((1, BN_ROWS, GW), lambda i, g=g: (g, i, 0))


def _pass_a(h6, a6, w1, vecs, batch3):
    def body(h0, h1, h2, h3, h4, h5, a0, a1, a2, a3, a4, a5,
             w1_r, vecs_r, b3_r, z1_r, st_r, s_r):
        i = pl.program_id(0)
        hin = jnp.concatenate([h[0] for h in (h0, h1, h2, h3, h4, h5)],
                              axis=1)
        aggr = jnp.concatenate([a[0] for a in (a0, a1, a2, a3, a4, a5)],
                               axis=1)
        z = hin * vecs_r[1, 0] + aggr
        z1 = jnp.dot(z, w1_r[...], preferred_element_type=jnp.float32)
        z1 = z1 + vecs_r[0, :]
        z1_r[...] = z1
        oh = _onehot(b3_r[0, 0, :])
        sacc = lax.dot_general(oh, hin, (((0,), (0,)), ((), ())),
                               preferred_element_type=jnp.float32)
        rowid = i * BN_ROWS + lax.broadcasted_iota(jnp.int32, (BN_ROWS, 1), 0)
        st = _stats8(z1, rowid < N)

        @pl.when(i == 0)
        def _():
            st_r[...] = st
            s_r[...] = sacc

        @pl.when(i > 0)
        def _():
            st_r[...] = st_r[...] + st
            s_r[...] = s_r[...] + sacc

    return pl.pallas_call(
        body,
        grid=(NBLK,),
        in_specs=[_g6_spec(g) for g in range(NG)] * 2 + [
            pl.BlockSpec((DP, HP), lambda i: (0, 0)),
            pl.BlockSpec((8, HP), lambda i: (0, 0)),
            pl.BlockSpec((1, 1, BN_ROWS), lambda i: (i, 0, 0)),
        ],
        out_specs=[
            pl.BlockSpec((BN_ROWS, HP), lambda i: (i, 0)),
            pl.BlockSpec((8, HP), lambda i: (0, 0)),
            pl.BlockSpec((G, DP), lambda i: (0, 0)),
        ],
        out_shape=[
            jax.ShapeDtypeStruct((NP, HP), jnp.float32),
            jax.ShapeDtypeStruct((8, HP), jnp.float32),
            jax.ShapeDtypeStruct((G, DP), jnp.float32),
        ],
    )(*([h6] * NG), *([a6] * NG), w1, vecs, batch3)


def _pass_b(z1, st1, gb, w2, vecs):
    def body(z1_r, st_r, gb_r, w2_r, vecs_r, z2_r, st2_r):
        i = pl.program_id(0)
        scale, shift = _scale_shift(st_r[...], gb_r[0, :], gb_r[1, :])
        a = jnp.maximum(z1_r[...] * scale + shift, 0.0)
        z2 = jnp.dot(a, w2_r[...], preferred_element_type=jnp.float32)
        z2 = z2 + vecs_r[0, :]
        z2_r[...] = z2
        rowid = i * BN_ROWS + lax.broadcasted_iota(jnp.int32, (BN_ROWS, 1), 0)
        st = _stats8(z2, rowid < N)

        @pl.when(i == 0)
        def _():
            st2_r[...] = st

        @pl.when(i > 0)
        def _():
            st2_r[...] = st2_r[...] + st

    return pl.pallas_call(
        body,
        grid=(NBLK,),
        in_specs=[
            pl.BlockSpec((BN_ROWS, HP), lambda i: (i, 0)),
            pl.BlockSpec((8, HP), lambda i: (0, 0)),
            pl.BlockSpec((8, HP), lambda i: (0, 0)),
            pl.BlockSpec((HP, DP), lambda i: (0, 0)),
            pl.BlockSpec((8, DP), lambda i: (0, 0)),
        ],
        out_specs=[
            pl.BlockSpec((BN_ROWS, DP), lambda i: (i, 0)),
            pl.BlockSpec((8, DP), lambda i: (0, 0)),
        ],
        out_shape=[
            jax.ShapeDtypeStruct((NP, DP), jnp.float32),
            jax.ShapeDtypeStruct((8, DP), jnp.float32),
        ],
    )(z1, st1, gb, w2, vecs)


def _pass_c(z2, st2, gb, vn, batch3):
    def body(z2_r, st_r, gb_r, vn_r, b3_r, *outs):
        scale, shift = _scale_shift(st_r[...], gb_r[0, :], gb_r[1, :])
        h = jnp.maximum(z2_r[...] * scale + shift, 0.0)
        oh = _onehot(b3_r[0, 0, :])
        hin = h + jnp.dot(oh, vn_r[...], preferred_element_type=jnp.float32)
        for g in range(NG):
            outs[g][...] = hin[:, g * GW:(g + 1) * GW]

    return pl.pallas_call(
        body,
        grid=(NBLK,),
        in_specs=[
            pl.BlockSpec((BN_ROWS, DP), lambda i: (i, 0)),
            pl.BlockSpec((8, DP), lambda i: (0, 0)),
            pl.BlockSpec((8, DP), lambda i: (0, 0)),
            pl.BlockSpec((G, DP), lambda i: (0, 0)),
            pl.BlockSpec((1, 1, BN_ROWS), lambda i: (i, 0, 0)),
        ],
        out_specs=[_grp_spec() for _ in range(NG)],
        out_shape=[jax.ShapeDtypeStruct((NP, GW), jnp.float32)
                   for _ in range(NG)],
    )(z2, st2, gb, vn, batch3)


def _pass_c_last(z2, st2, gb, batch3):
    def body(z2_r, st_r, gb_r, b3_r, s5_r):
        i = pl.program_id(0)
        scale, shift = _scale_shift(st_r[...], gb_r[0, :], gb_r[1, :])
        h = z2_r[...] * scale + shift
        oh = _onehot(b3_r[0, 0, :])
        sacc = lax.dot_general(oh, h, (((0,), (0,)), ((), ())),
                               preferred_element_type=jnp.float32)

        @pl.when(i == 0)
        def _():
            s5_r[...] = sacc

        @pl.when(i > 0)
        def _():
            s5_r[...] = s5_r[...] + sacc

    return pl.pallas_call(
        body,
        grid=(NBLK,),
        in_specs=[
            pl.BlockSpec((BN_ROWS, DP), lambda i: (i, 0)),
            pl.BlockSpec((8, DP), lambda i: (0, 0)),
            pl.BlockSpec((8, DP), lambda i: (0, 0)),
            pl.BlockSpec((1, 1, BN_ROWS), lambda i: (i, 0, 0)),
        ],
        out_specs=[pl.BlockSpec((G, DP), lambda i: (0, 0))],
        out_shape=[jax.ShapeDtypeStruct((G, DP), jnp.float32)],
    )(z2, st2, gb, batch3)


def _vn_mlp(s_in, vn_in, w1, vecs1, w2, vecs2):
    def body(s_r, vn_r, w1_r, v1_r, w2_r, v2_r, out_r):
        svn = s_r[...] + vn_r[...]
        u = jnp.dot(svn, w1_r[...], preferred_element_type=jnp.float32)
        u = u + v1_r[0, :]
        m = jnp.mean(u, axis=0)
        v = jnp.mean(u * u, axis=0) - m * m
        u = jnp.maximum(v1_r[1, :] * (u - m) * lax.rsqrt(v + 1e-5)
                        + v1_r[2, :], 0.0)
        u2 = jnp.dot(u, w2_r[...], preferred_element_type=jnp.float32)
        u2 = u2 + v2_r[0, :]
        m2 = jnp.mean(u2, axis=0)
        vv2 = jnp.mean(u2 * u2, axis=0) - m2 * m2
        out_r[...] = jnp.maximum(
            v2_r[1, :] * (u2 - m2) * lax.rsqrt(vv2 + 1e-5) + v2_r[2, :], 0.0)

    return pl.pallas_call(
        body,
        out_shape=jax.ShapeDtypeStruct((G, DP), jnp.float32),
    )(s_in, vn_in, w1, vecs1, w2, vecs2)


def _final(ss, batch3, wp, vecs):
    def body(ss_r, b3_r, wp_r, vecs_r, emb_r, log_r):
        pooled = jnp.sum(ss_r[...], axis=0)
        cnt = jnp.zeros((G,), jnp.float32)
        for i in range(NBLK):
            oh = _onehot(b3_r[i, 0, :])
            cnt = cnt + jnp.sum(oh, axis=0)
        emb = pooled / jnp.maximum(cnt, 1.0)[:, None]
        emb_r[...] = emb
        log_r[...] = jnp.dot(emb, wp_r[...],
                             preferred_element_type=jnp.float32) + vecs_r[0, :]

    return pl.pallas_call(
        body,
        out_shape=[
            jax.ShapeDtypeStruct((G, DP), jnp.float32),
            jax.ShapeDtypeStruct((G, T), jnp.float32),
        ],
    )(ss, batch3, wp, vecs)


def _pad_cols(a, w):
    return jnp.pad(a, [(0, 0)] * (a.ndim - 1) + [(0, w - a.shape[-1])])


def _vec8(*rows):
    w = rows[0].shape[-1]
    out = jnp.zeros((8, w), jnp.float32)
    for i, r in enumerate(rows):
        out = out.at[i, :].set(r)
    return out


def _groups(a):
    return [a[..., g * GW:(g + 1) * GW] for g in range(NG)]


def kernel(x, edge_index, edge_attr, batch, atom_tables, bond_tables, eps,
           W1, b1, bnm_g, bnm_b, W2, b2, bn_g, bn_b, Wv1, bv1, bnv1_g,
           bnv1_b, Wv2, bv2, bnv2_g, bnv2_b, Wp, bp):
    f32 = jnp.float32
    i32 = jnp.int32

    atab = _pad_cols(atom_tables.astype(f32).reshape(9 * 128, D), DP)
    at6 = jnp.stack(_groups(atab), axis=0)

    xi = x.astype(i32) + 128 * jnp.arange(9, dtype=i32)[None, :]
    xi = jnp.pad(xi, ((0, NP - N), (0, 0)))
    xidxT = jnp.transpose(xi.T.reshape(9, 16, ROWS_PER_SUB), (1, 0, 2))

    cidx = jnp.arange(512, dtype=i32)
    bt = bond_tables.astype(f32)
    ctab = (bt[:, 0, cidx >> 6, :] + bt[:, 1, (cidx >> 3) & 7, :]
            + bt[:, 2, cidx & 7, :])
    ctab = _pad_cols(ctab, DP)
    ctab = jnp.concatenate(
        [ctab, jnp.full((L, CC - 512, DP), -1e30, f32)], axis=1)
    ct6 = [jnp.stack(_groups(ctab[l]), axis=0) for l in range(L)]

    ea = edge_attr.astype(i32)
    code = ea[:, 0] * 64 + ea[:, 1] * 8 + ea[:, 2]
    srcT = jnp.pad(edge_index[0].astype(i32),
                   (0, EP - E)).reshape(16, ECHUNKS, CH)
    dstT = jnp.pad(edge_index[1].astype(i32),
                   (0, EP - E)).reshape(16, ECHUNKS, CH)
    codeT = jnp.pad(code, (0, EP - E),
                    constant_values=512).reshape(16, ECHUNKS, CH)

    batch_p = jnp.pad(batch.astype(i32), (0, NP - N), constant_values=G)
    batch3 = batch_p.reshape(NBLK, 1, BN_ROWS)

    W1p = _pad_cols(jnp.pad(W1.astype(f32), ((0, 0), (0, DP - D), (0, 0))), HP)
    W2p = _pad_cols(jnp.pad(W2.astype(f32), ((0, 0), (0, HP - H2), (0, 0))), DP)
    Wv1p = _pad_cols(jnp.pad(Wv1.astype(f32), ((0, 0), (0, DP - D), (0, 0))), HP)
    Wv2p = _pad_cols(jnp.pad(Wv2.astype(f32), ((0, 0), (0, HP - H2), (0, 0))), DP)
    Wpp = jnp.pad(Wp.astype(f32), ((0, DP - D), (0, 0)))

    b1p = _pad_cols(b1.astype(f32), HP)
    b2p = _pad_cols(b2.astype(f32), DP)
    one = jnp.ones((HP,), f32)
    vecsA = [_vec8(b1p[l], one * (1.0 + eps[l])) for l in range(L)]
    gb1 = [_vec8(_pad_cols(bnm_g.astype(f32), HP)[l],
                 _pad_cols(bnm_b.astype(f32), HP)[l]) for l in range(L)]
    vecsB = [_vec8(b2p[l]) for l in range(L)]
    gb2 = [_vec8(_pad_cols(bn_g.astype(f32), DP)[l],
                 _pad_cols(bn_b.astype(f32), DP)[l]) for l in range(L)]
    vecsV1 = [_vec8(_pad_cols(bv1.astype(f32), HP)[l],
                    _pad_cols(bnv1_g.astype(f32), HP)[l],
                    _pad_cols(bnv1_b.astype(f32), HP)[l]) for l in range(L - 1)]
    vecsV2 = [_vec8(_pad_cols(bv2.astype(f32), DP)[l],
                    _pad_cols(bnv2_g.astype(f32), DP)[l],
                    _pad_cols(bnv2_b.astype(f32), DP)[l]) for l in range(L - 1)]
    vecsP = _vec8(bp.astype(f32))

    h6 = _sc_embed(at6, xidxT)
    vn = jnp.zeros((G, DP), f32)
    s_list = []
    s5 = None
    for l in range(L):
        a6 = _sc_edge_all(h6, ct6[l], srcT, dstT, codeT)
        z1, st1, s_l = _pass_a(h6, a6, W1p[l], vecsA[l], batch3)
        s_list.append(s_l)
        z2, st2 = _pass_b(z1, st1, gb1[l], W2p[l], vecsB[l])
        if l < L - 1:
            vn = _vn_mlp(s_l, vn, Wv1p[l], vecsV1[l], Wv2p[l], vecsV2[l])
            h6 = jnp.stack(_pass_c(z2, st2, gb2[l], vn, batch3), axis=0)
        else:
            (s5,) = _pass_c_last(z2, st2, gb2[l], batch3)

    ss = jnp.stack(s_list + [s5], axis=0)
    emb_p, logits = _final(ss, batch3, Wpp, vecsP)
    return emb_p[:, :D], logits


# speedup vs baseline: 3.2608x; 1.2138x over previous
import jax
import jax.numpy as jnp
from jax import lax
from jax.experimental import pallas as pl
from jax.experimental.pallas import tpu as pltpu
from jax.experimental.pallas import tpu_sc as plsc

N = 10000
NP = 10240
E = 160000
EP = 163840
D = 300
DP = 384
GW = 64
NG = 6
H2 = 600
HP = 768
G = 128
L = 5
T = 128
CC = 640
BN_ROWS = 1024
NBLK = NP // BN_ROWS
CH = 128
EDGES_PER_SUB = EP // 16
ECHUNKS = EDGES_PER_SUB // CH
NCHUNK2 = ECHUNKS // 2
ROWS_PER_SUB = NP // 16
RCHUNKS = ROWS_PER_SUB // CH

def _sc_params():
    return dict(
        mesh=plsc.VectorSubcoreMesh(core_axis_name="c", subcore_axis_name="s",
                                    num_cores=2, num_subcores=16),
        compiler_params=pltpu.CompilerParams(use_tc_tiling_on_sc=False),
    )


def _zero_rows(buf):
    z = jnp.zeros((16,), jnp.float32)

    def row(r, _):
        for j in range(GW // 16):
            buf[r, pl.ds(j * 16, 16)] = z
        return 0

    lax.fori_loop(0, CH, row, 0)


def _sc_embed(at6, xidxT):
    def body(at_r, xidx_r, out_r, idxall, g0, g1, accbuf, sem0, sem1):
        c = lax.axis_index("c")
        s = lax.axis_index("s")
        gb = (g0, g1)
        sems = (sem0, sem1)

        pltpu.sync_copy(xidx_r.at[s], idxall)

        def per_group(gi, _):
            g = c * 3 + gi
            tab_ref = at_r.at[g]

            def per_chunk(k, _):
                rowbase = (s * RCHUNKS + k) * CH

                def idxk(f):
                    return idxall.at[f, pl.ds(k * CH, CH)]

                pltpu.async_copy(tab_ref.at[idxk(0)], gb[0], sems[0])
                for f in range(9):
                    p = f % 2
                    if f < 8:
                        q = 1 - p
                        pltpu.async_copy(tab_ref.at[idxk(f + 1)], gb[q],
                                         sems[q])
                    pltpu.make_async_copy(tab_ref.at[idxk(f)], gb[p],
                                          sems[p]).wait()
                    src = gb[p]
                    if f == 0:
                        def rowcp(r, _):
                            for rr in range(2):
                                for j in range(GW // 16):
                                    sl = pl.ds(j * 16, 16)
                                    accbuf[2 * r + rr, sl] = src[2 * r + rr, sl]
                            return 0
                        lax.fori_loop(0, CH // 2, rowcp, 0)
                    else:
                        def rowadd(r, _):
                            for rr in range(2):
                                for j in range(GW // 16):
                                    sl = pl.ds(j * 16, 16)
                                    accbuf[2 * r + rr, sl] = (
                                        accbuf[2 * r + rr, sl]
                                        + src[2 * r + rr, sl])
                            return 0
                        lax.fori_loop(0, CH // 2, rowadd, 0)
                pltpu.sync_copy(accbuf, out_r.at[g, pl.ds(rowbase, CH)])
                return 0

            lax.fori_loop(0, RCHUNKS, per_chunk, 0)
            return 0

        lax.fori_loop(0, 3, per_group, 0)

    f = pl.kernel(
        body,
        out_type=jax.ShapeDtypeStruct((NG, NP, GW), jnp.float32),
        scratch_types=[
            pltpu.VMEM((9, ROWS_PER_SUB), jnp.int32),
            pltpu.VMEM((CH, GW), jnp.float32),
            pltpu.VMEM((CH, GW), jnp.float32),
            pltpu.VMEM((CH, GW), jnp.float32),
            pltpu.SemaphoreType.DMA,
            pltpu.SemaphoreType.DMA,
        ],
        **_sc_params(),
    )
    return f(at6, xidxT)


def _sc_edge_all(h6, ct6, srcT, dstT, codeT):
    def body(h_r, ct_r, src_r, dst_r, code_r, out_r,
             srcall, dstall, codeall,
             hb0, hb1, eb0, eb1, sb0, tmpbuf, ctloc, aggr_sh,
             semh0, semh1, seme0, seme1):
        c = lax.axis_index("c")
        s = lax.axis_index("s")
        hb = (hb0, hb1)
        eb = (eb0, eb1)
        semh = (semh0, semh1)
        seme = (seme0, seme1)

        pltpu.sync_copy(src_r.at[s], srcall)
        pltpu.sync_copy(dst_r.at[s], dstall)
        pltpu.sync_copy(code_r.at[s], codeall)
        _zero_rows(tmpbuf)

        def per_group(gi, _):
            g = c * 3 + gi
            h_ref = h_r.at[g]
            @pl.when(s < CC // CH)
            def _():
                pltpu.sync_copy(ct_r.at[g, pl.ds(s * CH, CH)], eb0)
                pltpu.sync_copy(eb0, ctloc.at[pl.ds(s * CH, CH)])
            for k in range(RCHUNKS):
                pltpu.sync_copy(tmpbuf,
                                aggr_sh.at[pl.ds((s * RCHUNKS + k) * CH, CH)])
            plsc.subcore_barrier()

            def sidx(k):
                return srcall.at[k]

            def cidx(k):
                return codeall.at[k]

            def didx(k):
                return dstall.at[k]

            def fire(k, p):
                pltpu.async_copy(h_ref.at[sidx(k)], hb[p], semh[p])
                pltpu.async_copy(ctloc.at[cidx(k)], eb[p], seme[p])

            fire(0, 0)

            def loop(k2, _):
                for p in (0, 1):
                    k = k2 * 2 + p
                    nk = k + 1

                    @pl.when(nk < ECHUNKS)
                    def _():
                        q = 1 - p
                        pltpu.async_copy(h_ref.at[sidx(nk)], hb[q], semh[q])
                        pltpu.async_copy(ctloc.at[cidx(nk)], eb[q], seme[q])

                    pltpu.make_async_copy(h_ref.at[sidx(k)], hb[p],
                                          semh[p]).wait()
                    pltpu.make_async_copy(ctloc.at[cidx(k)], eb[p],
                                          seme[p]).wait()
                    hp, ep = hb[p], eb[p]

                    def row(r, _):
                        for rr in range(2):
                            for j in range(GW // 16):
                                sl = pl.ds(j * 16, 16)
                                ep[2 * r + rr, sl] = jnp.maximum(
                                    hp[2 * r + rr, sl]
                                    + ep[2 * r + rr, sl], 0.0)
                        return 0

                    lax.fori_loop(0, CH // 2, row, 0)
                    pltpu.sync_copy(ep, aggr_sh.at[didx(k)], add=True)
                return 0

            lax.fori_loop(0, ECHUNKS // 2, loop, 0)
            plsc.subcore_barrier()
            for k in range(RCHUNKS):
                r0 = (s * RCHUNKS + k) * CH
                pltpu.sync_copy(aggr_sh.at[pl.ds(r0, CH)], sb0)
                pltpu.sync_copy(sb0, out_r.at[g, pl.ds(r0, CH)])
            return 0

        lax.fori_loop(0, 3, per_group, 0)

    f = pl.kernel(
        body,
        out_type=jax.ShapeDtypeStruct((NG, NP, GW), jnp.float32),
        scratch_types=[
            pltpu.VMEM((ECHUNKS, CH), jnp.int32),
            pltpu.VMEM((ECHUNKS, CH), jnp.int32),
            pltpu.VMEM((ECHUNKS, CH), jnp.int32),
            pltpu.VMEM((CH, GW), jnp.float32),
            pltpu.VMEM((CH, GW), jnp.float32),
            pltpu.VMEM((CH, GW), jnp.float32),
            pltpu.VMEM((CH, GW), jnp.float32),
            pltpu.VMEM((CH, GW), jnp.float32),
            pltpu.VMEM((CH, GW), jnp.float32),
            pltpu.VMEM_SHARED((CC, GW), jnp.float32),
            pltpu.VMEM_SHARED((NP, GW), jnp.float32),
            pltpu.SemaphoreType.DMA,
            pltpu.SemaphoreType.DMA,
            pltpu.SemaphoreType.DMA,
            pltpu.SemaphoreType.DMA,
        ],
        **_sc_params(),
    )
    return f(h6, ct6, srcT, dstT, codeT)


def _onehot(b):
    return (b[:, None] == lax.broadcasted_iota(jnp.int32, (BN_ROWS, G), 1)
            ).astype(jnp.float32)


def _stats8(x, mask):
    xm = jnp.where(mask, x, 0.0)
    s0 = jnp.sum(xm, axis=0, keepdims=True)
    s1 = jnp.sum(xm * xm, axis=0, keepdims=True)
    return jnp.concatenate(
        [s0, s1, jnp.zeros((6, x.shape[1]), jnp.float32)], axis=0)


def _scale_shift(stats, g, b):
    m = stats[0, :] * (1.0 / N)
    v = stats[1, :] * (1.0 / N) - m * m
    scale = g * lax.rsqrt(v + 1e-5)
    return scale, b - m * scale


def _grp_spec():
    return pl.BlockSpec((BN_ROWS, GW), lambda i: (i, 0))


def _g6_spec(g):
    return pl.BlockSpec((1, BN_ROWS, GW), lambda i, g=g: (g, i, 0))


def _pass_a(h6, a6, w1, vecs, batch3):
    def body(h0, h1, h2, h3, h4, h5, a0, a1, a2, a3, a4, a5,
             w1_r, vecs_r, b3_r, z1_r, st_r, s_r):
        i = pl.program_id(0)
        hin = jnp.concatenate([h[0] for h in (h0, h1, h2, h3, h4, h5)],
                              axis=1)
        aggr = jnp.concatenate([a[0] for a in (a0, a1, a2, a3, a4, a5)],
                               axis=1)
        z = hin * vecs_r[1, 0] + aggr
        z1 = jnp.dot(z, w1_r[...], preferred_element_type=jnp.float32)
        z1 = z1 + vecs_r[0, :]
        z1_r[...] = z1
        oh = _onehot(b3_r[0, 0, :])
        sacc = lax.dot_general(oh, hin, (((0,), (0,)), ((), ())),
                               preferred_element_type=jnp.float32)
        rowid = i * BN_ROWS + lax.broadcasted_iota(jnp.int32, (BN_ROWS, 1), 0)
        st = _stats8(z1, rowid < N)

        @pl.when(i == 0)
        def _():
            st_r[...] = st
            s_r[...] = sacc

        @pl.when(i > 0)
        def _():
            st_r[...] = st_r[...] + st
            s_r[...] = s_r[...] + sacc

    return pl.pallas_call(
        body,
        grid=(NBLK,),
        in_specs=[_g6_spec(g) for g in range(NG)] * 2 + [
            pl.BlockSpec((DP, HP), lambda i: (0, 0)),
            pl.BlockSpec((8, HP), lambda i: (0, 0)),
            pl.BlockSpec((1, 1, BN_ROWS), lambda i: (i, 0, 0)),
        ],
        out_specs=[
            pl.BlockSpec((BN_ROWS, HP), lambda i: (i, 0)),
            pl.BlockSpec((8, HP), lambda i: (0, 0)),
            pl.BlockSpec((G, DP), lambda i: (0, 0)),
        ],
        out_shape=[
            jax.ShapeDtypeStruct((NP, HP), jnp.float32),
            jax.ShapeDtypeStruct((8, HP), jnp.float32),
            jax.ShapeDtypeStruct((G, DP), jnp.float32),
        ],
    )(*([h6] * NG), *([a6] * NG), w1, vecs, batch3)


def _pass_b(z1, st1, gb, w2, vecs):
    def body(z1_r, st_r, gb_r, w2_r, vecs_r, z2_r, st2_r):
        i = pl.program_id(0)
        scale, shift = _scale_shift(st_r[...], gb_r[0, :], gb_r[1, :])
        a = jnp.maximum(z1_r[...] * scale + shift, 0.0)
        z2 = jnp.dot(a, w2_r[...], preferred_element_type=jnp.float32)
        z2 = z2 + vecs_r[0, :]
        z2_r[...] = z2
        rowid = i * BN_ROWS + lax.broadcasted_iota(jnp.int32, (BN_ROWS, 1), 0)
        st = _stats8(z2, rowid < N)

        @pl.when(i == 0)
        def _():
            st2_r[...] = st

        @pl.when(i > 0)
        def _():
            st2_r[...] = st2_r[...] + st

    return pl.pallas_call(
        body,
        grid=(NBLK,),
        in_specs=[
            pl.BlockSpec((BN_ROWS, HP), lambda i: (i, 0)),
            pl.BlockSpec((8, HP), lambda i: (0, 0)),
            pl.BlockSpec((8, HP), lambda i: (0, 0)),
            pl.BlockSpec((HP, DP), lambda i: (0, 0)),
            pl.BlockSpec((8, DP), lambda i: (0, 0)),
        ],
        out_specs=[
            pl.BlockSpec((BN_ROWS, DP), lambda i: (i, 0)),
            pl.BlockSpec((8, DP), lambda i: (0, 0)),
        ],
        out_shape=[
            jax.ShapeDtypeStruct((NP, DP), jnp.float32),
            jax.ShapeDtypeStruct((8, DP), jnp.float32),
        ],
    )(z1, st1, gb, w2, vecs)


def _pass_c(z2, st2, gb, vn, batch3):
    def body(z2_r, st_r, gb_r, vn_r, b3_r, *outs):
        scale, shift = _scale_shift(st_r[...], gb_r[0, :], gb_r[1, :])
        h = jnp.maximum(z2_r[...] * scale + shift, 0.0)
        oh = _onehot(b3_r[0, 0, :])
        hin = h + jnp.dot(oh, vn_r[...], preferred_element_type=jnp.float32)
        for g in range(NG):
            outs[g][...] = hin[:, g * GW:(g + 1) * GW]

    return pl.pallas_call(
        body,
        grid=(NBLK,),
        in_specs=[
            pl.BlockSpec((BN_ROWS, DP), lambda i: (i, 0)),
            pl.BlockSpec((8, DP), lambda i: (0, 0)),
            pl.BlockSpec((8, DP), lambda i: (0, 0)),
            pl.BlockSpec((G, DP), lambda i: (0, 0)),
            pl.BlockSpec((1, 1, BN_ROWS), lambda i: (i, 0, 0)),
        ],
        out_specs=[_grp_spec() for _ in range(NG)],
        out_shape=[jax.ShapeDtypeStruct((NP, GW), jnp.float32)
                   for _ in range(NG)],
    )(z2, st2, gb, vn, batch3)


def _pass_c_last(z2, st2, gb, batch3):
    def body(z2_r, st_r, gb_r, b3_r, s5_r):
        i = pl.program_id(0)
        scale, shift = _scale_shift(st_r[...], gb_r[0, :], gb_r[1, :])
        h = z2_r[...] * scale + shift
        oh = _onehot(b3_r[0, 0, :])
        sacc = lax.dot_general(oh, h, (((0,), (0,)), ((), ())),
                               preferred_element_type=jnp.float32)

        @pl.when(i == 0)
        def _():
            s5_r[...] = sacc

        @pl.when(i > 0)
        def _():
            s5_r[...] = s5_r[...] + sacc

    return pl.pallas_call(
        body,
        grid=(NBLK,),
        in_specs=[
            pl.BlockSpec((BN_ROWS, DP), lambda i: (i, 0)),
            pl.BlockSpec((8, DP), lambda i: (0, 0)),
            pl.BlockSpec((8, DP), lambda i: (0, 0)),
            pl.BlockSpec((1, 1, BN_ROWS), lambda i: (i, 0, 0)),
        ],
        out_specs=[pl.BlockSpec((G, DP), lambda i: (0, 0))],
        out_shape=[jax.ShapeDtypeStruct((G, DP), jnp.float32)],
    )(z2, st2, gb, batch3)


def _vn_mlp(s_in, vn_in, w1, vecs1, w2, vecs2):
    def body(s_r, vn_r, w1_r, v1_r, w2_r, v2_r, out_r):
        svn = s_r[...] + vn_r[...]
        u = jnp.dot(svn, w1_r[...], preferred_element_type=jnp.float32)
        u = u + v1_r[0, :]
        m = jnp.mean(u, axis=0)
        v = jnp.mean(u * u, axis=0) - m * m
        u = jnp.maximum(v1_r[1, :] * (u - m) * lax.rsqrt(v + 1e-5)
                        + v1_r[2, :], 0.0)
        u2 = jnp.dot(u, w2_r[...], preferred_element_type=jnp.float32)
        u2 = u2 + v2_r[0, :]
        m2 = jnp.mean(u2, axis=0)
        vv2 = jnp.mean(u2 * u2, axis=0) - m2 * m2
        out_r[...] = jnp.maximum(
            v2_r[1, :] * (u2 - m2) * lax.rsqrt(vv2 + 1e-5) + v2_r[2, :], 0.0)

    return pl.pallas_call(
        body,
        out_shape=jax.ShapeDtypeStruct((G, DP), jnp.float32),
    )(s_in, vn_in, w1, vecs1, w2, vecs2)


def _final(ss, batch3, wp, vecs):
    def body(ss_r, b3_r, wp_r, vecs_r, emb_r, log_r):
        pooled = jnp.sum(ss_r[...], axis=0)
        cnt = jnp.zeros((G,), jnp.float32)
        for i in range(NBLK):
            oh = _onehot(b3_r[i, 0, :])
            cnt = cnt + jnp.sum(oh, axis=0)
        emb = pooled / jnp.maximum(cnt, 1.0)[:, None]
        emb_r[...] = emb
        log_r[...] = jnp.dot(emb, wp_r[...],
                             preferred_element_type=jnp.float32) + vecs_r[0, :]

    return pl.pallas_call(
        body,
        out_shape=[
            jax.ShapeDtypeStruct((G, DP), jnp.float32),
            jax.ShapeDtypeStruct((G, T), jnp.float32),
        ],
    )(ss, batch3, wp, vecs)


def _pad_cols(a, w):
    return jnp.pad(a, [(0, 0)] * (a.ndim - 1) + [(0, w - a.shape[-1])])


def _vec8(*rows):
    w = rows[0].shape[-1]
    out = jnp.zeros((8, w), jnp.float32)
    for i, r in enumerate(rows):
        out = out.at[i, :].set(r)
    return out


def _groups(a):
    return [a[..., g * GW:(g + 1) * GW] for g in range(NG)]


def kernel(x, edge_index, edge_attr, batch, atom_tables, bond_tables, eps,
           W1, b1, bnm_g, bnm_b, W2, b2, bn_g, bn_b, Wv1, bv1, bnv1_g,
           bnv1_b, Wv2, bv2, bnv2_g, bnv2_b, Wp, bp):
    f32 = jnp.float32
    i32 = jnp.int32

    atab = _pad_cols(atom_tables.astype(f32).reshape(9 * 128, D), DP)
    at6 = jnp.stack(_groups(atab), axis=0)

    xi = x.astype(i32) + 128 * jnp.arange(9, dtype=i32)[None, :]
    xi = jnp.pad(xi, ((0, NP - N), (0, 0)))
    xidxT = jnp.transpose(xi.T.reshape(9, 16, ROWS_PER_SUB), (1, 0, 2))

    cidx = jnp.arange(512, dtype=i32)
    bt = bond_tables.astype(f32)
    ctab = (bt[:, 0, cidx >> 6, :] + bt[:, 1, (cidx >> 3) & 7, :]
            + bt[:, 2, cidx & 7, :])
    ctab = _pad_cols(ctab, DP)
    ctab = jnp.concatenate(
        [ctab, jnp.full((L, CC - 512, DP), -1e30, f32)], axis=1)
    ct6 = [jnp.stack(_groups(ctab[l]), axis=0) for l in range(L)]

    ea = edge_attr.astype(i32)
    code = ea[:, 0] * 64 + ea[:, 1] * 8 + ea[:, 2]
    srcT = jnp.pad(edge_index[0].astype(i32),
                   (0, EP - E)).reshape(16, ECHUNKS, CH)
    dstT = jnp.pad(edge_index[1].astype(i32),
                   (0, EP - E)).reshape(16, ECHUNKS, CH)
    codeT = jnp.pad(code, (0, EP - E),
                    constant_values=512).reshape(16, ECHUNKS, CH)

    batch_p = jnp.pad(batch.astype(i32), (0, NP - N), constant_values=G)
    batch3 = batch_p.reshape(NBLK, 1, BN_ROWS)

    W1p = _pad_cols(jnp.pad(W1.astype(f32), ((0, 0), (0, DP - D), (0, 0))), HP)
    W2p = _pad_cols(jnp.pad(W2.astype(f32), ((0, 0), (0, HP - H2), (0, 0))), DP)
    Wv1p = _pad_cols(jnp.pad(Wv1.astype(f32), ((0, 0), (0, DP - D), (0, 0))), HP)
    Wv2p = _pad_cols(jnp.pad(Wv2.astype(f32), ((0, 0), (0, HP - H2), (0, 0))), DP)
    Wpp = jnp.pad(Wp.astype(f32), ((0, DP - D), (0, 0)))

    b1p = _pad_cols(b1.astype(f32), HP)
    b2p = _pad_cols(b2.astype(f32), DP)
    one = jnp.ones((HP,), f32)
    vecsA = [_vec8(b1p[l], one * (1.0 + eps[l])) for l in range(L)]
    gb1 = [_vec8(_pad_cols(bnm_g.astype(f32), HP)[l],
                 _pad_cols(bnm_b.astype(f32), HP)[l]) for l in range(L)]
    vecsB = [_vec8(b2p[l]) for l in range(L)]
    gb2 = [_vec8(_pad_cols(bn_g.astype(f32), DP)[l],
                 _pad_cols(bn_b.astype(f32), DP)[l]) for l in range(L)]
    vecsV1 = [_vec8(_pad_cols(bv1.astype(f32), HP)[l],
                    _pad_cols(bnv1_g.astype(f32), HP)[l],
                    _pad_cols(bnv1_b.astype(f32), HP)[l]) for l in range(L - 1)]
    vecsV2 = [_vec8(_pad_cols(bv2.astype(f32), DP)[l],
                    _pad_cols(bnv2_g.astype(f32), DP)[l],
                    _pad_cols(bnv2_b.astype(f32), DP)[l]) for l in range(L - 1)]
    vecsP = _vec8(bp.astype(f32))

    h6 = _sc_embed(at6, xidxT)
    vn = jnp.zeros((G, DP), f32)
    s_list = []
    s5 = None
    for l in range(L):
        a6 = _sc_edge_all(h6, ct6[l], srcT, dstT, codeT)
        z1, st1, s_l = _pass_a(h6, a6, W1p[l], vecsA[l], batch3)
        s_list.append(s_l)
        z2, st2 = _pass_b(z1, st1, gb1[l], W2p[l], vecsB[l])
        if l < L - 1:
            vn = _vn_mlp(s_l, vn, Wv1p[l], vecsV1[l], Wv2p[l], vecsV2[l])
            h6 = jnp.stack(_pass_c(z2, st2, gb2[l], vn, batch3), axis=0)
        else:
            (s5,) = _pass_c_last(z2, st2, gb2[l], batch3)

    ss = jnp.stack(s_list + [s5], axis=0)
    emb_p, logits = _final(ss, batch3, Wpp, vecsP)
    return emb_p[:, :D], logits
